# Initial kernel scaffold; baseline (speedup 1.0000x reference)
import jax
import jax.numpy as jnp
from jax.experimental import pallas as pl


def kernel(x, edge_index, edge_attr, conv1_w, conv1_root, conv1_b, conv2_w, conv2_root, conv2_b, mlp1_w, mlp1_b, mlp2_w, mlp2_b):
    raise NotImplementedError("write your pallas kernel here")


# speedup vs baseline: 6.4533x; 6.4533x over previous
import functools

import jax
import jax.numpy as jnp
from jax import lax
from jax.experimental import pallas as pl
from jax.experimental.pallas import tpu as pltpu
from jax.experimental.pallas import tpu_sc as plsc

_NC = 2
_NS = 16
_NW = _NC * _NS
_L = 16

_D = 128
_S = 4


def _prep_body(n, a0_ref, a1_ref, src_ref, r1_ref, b1_ref, r2_ref, b2_ref):
    a0 = a0_ref[...]
    a1 = a1_ref[...]
    srcv = src_ref[...]
    for ks, r_ref, b_ref in ((3, r1_ref, b1_ref), (5, r2_ref, b2_ref)):
        v0 = a0 * (ks - 1)
        bot0 = jnp.floor(v0)
        f0 = v0 - bot0
        i0 = bot0.astype(jnp.int32)
        v1 = a1 * (ks - 1)
        bot1 = jnp.floor(v1)
        f1 = v1 - bot1
        i1 = bot1.astype(jnp.int32)
        ws, rs = [], []
        for s in range(_S):
            bit0 = s & 1
            bit1 = (s >> 1) & 1
            w0 = f0 if bit0 else 1.0 - f0
            w1 = f1 if bit1 else 1.0 - f1
            idx0 = jnp.clip(i0 + bit0, 0, ks - 1)
            idx1 = jnp.clip(i1 + bit1, 0, ks - 1)
            wi = idx0 + ks * idx1
            ws.append(w0 * w1)
            rs.append(wi * n + srcv)
        b_ref[...] = jnp.stack(ws)
        r_ref[...] = jnp.stack(rs)


def _prep(n, e, a0, a1, src2):
    rows, cols = a0.shape
    grid = 4
    cb = cols // grid
    in_spec = pl.BlockSpec((rows, cb), lambda i: (0, i))
    out_spec = pl.BlockSpec((_S, rows, cb), lambda i: (0, 0, i))
    f32 = jnp.float32
    return pl.pallas_call(
        functools.partial(_prep_body, n),
        grid=(grid,),
        in_specs=[in_spec, in_spec, in_spec],
        out_specs=[out_spec, out_spec, out_spec, out_spec],
        out_shape=[
            jax.ShapeDtypeStruct((_S, rows, cols), jnp.int32),
            jax.ShapeDtypeStruct((_S, rows, cols), f32),
            jax.ShapeDtypeStruct((_S, rows, cols), jnp.int32),
            jax.ShapeDtypeStruct((_S, rows, cols), f32),
        ],
    )(a0, a1, src2)


def _proj_body(x_ref, w_ref, o_ref):
    k = pl.program_id(1)
    o_ref[...] = jnp.dot(x_ref[...], w_ref[k],
                         preferred_element_type=jnp.float32)


def _proj(xin, wfull, bn=400):
    n = xin.shape[0]
    k1 = wfull.shape[0]
    nb = n // bn
    return pl.pallas_call(
        _proj_body,
        grid=(nb, k1),
        in_specs=[
            pl.BlockSpec((bn, _D), lambda i, k: (i, 0)),
            pl.BlockSpec((k1, _D, _D), lambda i, k: (0, 0, 0)),
        ],
        out_specs=pl.BlockSpec((bn, _D), lambda i, k: (k * nb + i, 0)),
        out_shape=jax.ShapeDtypeStruct((k1 * n, _D), jnp.float32),
    )(xin, wfull)


def _make_agg(n, e_total):
    epw = e_total // _NW
    B = 80
    nblk = epw // B
    G = B // _L
    chunk = (n // _NS) & ~7
    rem = n - _NS * chunk

    mesh = plsc.VectorSubcoreMesh(core_axis_name="c", subcore_axis_name="s",
                                  num_cores=_NC, num_subcores=_NS)

    out_type = [jax.ShapeDtypeStruct((_NC, n, _D), jnp.float32)]

    scratch = [
        pltpu.VMEM((_S * B,), jnp.int32),
        pltpu.VMEM((_S * B,), jnp.float32),
        pltpu.VMEM((B,), jnp.int32),
        pltpu.VMEM((B, _D), jnp.float32),
        pltpu.VMEM((B, _D), jnp.float32),
        pltpu.VMEM((B, _D), jnp.float32),
        pltpu.VMEM((B, _D), jnp.float32),
        pltpu.VMEM_SHARED((n, _D), jnp.float32),
        pltpu.SemaphoreType.DMA,
    ]

    def body(table, rows_h, bas_h, dst_h, z128, out_p,
             rix, bas, dstv, t0, t1, t2, t3, acc, sem):
        tbufs = (t0, t1, t2, t3)

        c = lax.axis_index("c")
        s = lax.axis_index("s")
        wid = s * _NC + c

        pltpu.sync_copy(z128.at[pl.ds(s * chunk, chunk)],
                        acc.at[pl.ds(s * chunk, chunk)])
        if rem:
            @pl.when(s == _NS - 1)
            def _zero_rem():
                pltpu.sync_copy(z128.at[pl.ds(_NS * chunk, rem)],
                                acc.at[pl.ds(_NS * chunk, rem)])
        plsc.subcore_barrier()

        base_e = wid * epw

        def block_body(j, carry):
            off = base_e + j * B
            for t in range(_S):
                pltpu.sync_copy(rows_h.at[pl.ds(t * e_total + off, B)],
                                rix.at[pl.ds(t * B, B)])
                pltpu.sync_copy(bas_h.at[pl.ds(t * e_total + off, B)],
                                bas.at[pl.ds(t * B, B)])
            pltpu.sync_copy(dst_h.at[pl.ds(off, B)], dstv)
            cps = [pltpu.async_copy(table.at[rix.at[pl.ds(t * B, B)]],
                                    tbufs[t], sem)
                   for t in range(_S)]
            for cp in cps:
                cp.wait()

            def g_body(g, carry2):
                bch = [bas[pl.ds(t * B + g * _L, _L)] for t in range(_S)]

                def l_body(l, carry3):
                    e = g * _L + l
                    lidx = jnp.zeros((_L,), jnp.int32) + l
                    bvs = [jnp.take(bch[t], lidx) for t in range(_S)]
                    for cch in range(_D // _L):
                        o = cch * _L
                        accv = None
                        for t in range(_S):
                            v = tbufs[t][e, pl.ds(o, _L)]
                            contrib = v * bvs[t]
                            accv = contrib if accv is None else accv + contrib
                        t0[e, pl.ds(o, _L)] = accv
                    return 0

                lax.fori_loop(0, _L, l_body, 0)
                return 0

            lax.fori_loop(0, G, g_body, 0)

            pltpu.sync_copy(t0, acc.at[dstv], add=True)
            return 0

        lax.fori_loop(0, nblk, block_body, 0)

        plsc.subcore_barrier()
        pltpu.sync_copy(acc.at[pl.ds(s * chunk, chunk)],
                        out_p.at[c, pl.ds(s * chunk, chunk)])
        if rem:
            @pl.when(s == _NS - 1)
            def _out_rem():
                pltpu.sync_copy(acc.at[pl.ds(_NS * chunk, rem)],
                                out_p.at[c, pl.ds(_NS * chunk, rem)])

    return pl.kernel(body, out_type=out_type, mesh=mesh,
                     scratch_types=scratch)


def _make_cnt(n, e_total):
    epw = e_total // _NW
    B = 80
    nblk = epw // B
    chunk = (n // _NS) & ~7
    rem = n - _NS * chunk

    mesh = plsc.VectorSubcoreMesh(core_axis_name="c", subcore_axis_name="s",
                                  num_cores=_NC, num_subcores=_NS)

    def body(dst_h, z16, out_c, dstv, ones, cacc, sem):
        c = lax.axis_index("c")
        s = lax.axis_index("s")
        wid = s * _NC + c

        pltpu.sync_copy(z16.at[pl.ds(s * chunk, chunk)],
                        cacc.at[pl.ds(s * chunk, chunk)])

        def ones_body(r, carry):
            for cch in range(_D // _L):
                ones[r, pl.ds(cch * _L, _L)] = jnp.zeros((_L,), jnp.float32) + 1.0
            return 0

        lax.fori_loop(0, B, ones_body, 0)
        if rem:
            @pl.when(s == _NS - 1)
            def _zero_rem():
                pltpu.sync_copy(z16.at[pl.ds(_NS * chunk, rem)],
                                cacc.at[pl.ds(_NS * chunk, rem)])
        plsc.subcore_barrier()

        base_e = wid * epw

        def block_body(j, carry):
            off = base_e + j * B
            pltpu.sync_copy(dst_h.at[pl.ds(off, B)], dstv)
            pltpu.sync_copy(ones, cacc.at[dstv], add=True)
            return 0

        lax.fori_loop(0, nblk, block_body, 0)

        plsc.subcore_barrier()
        pltpu.sync_copy(cacc.at[pl.ds(s * chunk, chunk)],
                        out_c.at[c, pl.ds(s * chunk, chunk)])
        if rem:
            @pl.when(s == _NS - 1)
            def _out_rem():
                pltpu.sync_copy(cacc.at[pl.ds(_NS * chunk, rem)],
                                out_c.at[c, pl.ds(_NS * chunk, rem)])

    return pl.kernel(
        body,
        out_type=[jax.ShapeDtypeStruct((_NC, n, _D), jnp.float32)],
        mesh=mesh,
        scratch_types=[
            pltpu.VMEM((B,), jnp.int32),
            pltpu.VMEM((B, _D), jnp.float32),
            pltpu.VMEM_SHARED((n, _D), jnp.float32),
            pltpu.SemaphoreType.DMA,
        ])


def _fin1_body(part_ref, cntp_ref, root_ref, b_ref, w_ref, o_ref):
    k = pl.program_id(1)
    aggsum = part_ref[0] + part_ref[1]
    cnt2 = cntp_ref[0] + cntp_ref[1]
    cnt = cnt2[:, 0:1]
    h = aggsum / jnp.maximum(cnt, 1.0) + root_ref[...] + b_ref[0]
    h = jnp.where(h > 0, h, jnp.exp(jnp.minimum(h, 0.0)) - 1.0)
    o_ref[...] = jnp.dot(h, w_ref[k], preferred_element_type=jnp.float32)


def _fin1proj2(part, cntp, root1t, b1, w2full, bn=400):
    n = root1t.shape[0]
    k1 = w2full.shape[0]
    nb = n // bn
    return pl.pallas_call(
        _fin1_body,
        grid=(nb, k1),
        in_specs=[
            pl.BlockSpec((_NC, bn, _D), lambda i, k: (0, i, 0)),
            pl.BlockSpec((_NC, bn, _D), lambda i, k: (0, i, 0)),
            pl.BlockSpec((bn, _D), lambda i, k: (i, 0)),
            pl.BlockSpec((1, _D), lambda i, k: (0, 0)),
            pl.BlockSpec((k1, _D, _D), lambda i, k: (0, 0, 0)),
        ],
        out_specs=pl.BlockSpec((bn, _D), lambda i, k: (k * nb + i, 0)),
        out_shape=jax.ShapeDtypeStruct((k1 * n, _D), jnp.float32),
    )(part, cntp, root1t, b1, w2full)


def _fin2_body(part_ref, cntp_ref, root_ref, b_ref, m1w_ref, m1b_ref,
               m2w_ref, m2b_ref, o_ref):
    aggsum = part_ref[0] + part_ref[1]
    cnt2 = cntp_ref[0] + cntp_ref[1]
    cnt = cnt2[:, 0:1]
    h = aggsum / jnp.maximum(cnt, 1.0) + root_ref[...] + b_ref[0]
    h = jnp.where(h > 0, h, jnp.exp(jnp.minimum(h, 0.0)) - 1.0)
    a = jnp.dot(h, m1w_ref[...], preferred_element_type=jnp.float32)
    a = jnp.maximum(a + m1b_ref[0], 0.0)
    o = jnp.dot(a, m2w_ref[...], preferred_element_type=jnp.float32)
    o_ref[...] = jnp.maximum(o + m2b_ref[0], 0.0)


def _fin2mlp(part, cntp, root2t, b2, m1w, m1b, m2w, m2b, bn=400):
    n = root2t.shape[0]
    co = m2w.shape[1]
    nb = n // bn
    return pl.pallas_call(
        _fin2_body,
        grid=(nb,),
        in_specs=[
            pl.BlockSpec((_NC, bn, _D), lambda i: (0, i, 0)),
            pl.BlockSpec((_NC, bn, _D), lambda i: (0, i, 0)),
            pl.BlockSpec((bn, _D), lambda i: (i, 0)),
            pl.BlockSpec((1, _D), lambda i: (0, 0)),
            pl.BlockSpec((_D, _D), lambda i: (0, 0)),
            pl.BlockSpec((1, _D), lambda i: (0, 0)),
            pl.BlockSpec((_D, co), lambda i: (0, 0)),
            pl.BlockSpec((1, co), lambda i: (0, 0)),
        ],
        out_specs=pl.BlockSpec((bn, co), lambda i: (i, 0)),
        out_shape=jax.ShapeDtypeStruct((n, co), jnp.float32),
    )(part, cntp, root2t, b2, m1w, m1b, m2w, m2b)


def kernel(x, edge_index, edge_attr, conv1_w, conv1_root, conv1_b,
           conv2_w, conv2_root, conv2_b, mlp1_w, mlp1_b, mlp2_w, mlp2_b):
    n = x.shape[0]
    e = edge_index.shape[1]
    k1 = conv1_w.shape[0]
    k2 = conv2_w.shape[0]

    src = edge_index[0]
    dst = edge_index[1]
    rows2d = 625
    cols2d = e // rows2d
    a0 = edge_attr[:, 0].reshape(rows2d, cols2d)
    a1 = edge_attr[:, 1].reshape(rows2d, cols2d)
    src2 = src.reshape(rows2d, cols2d)

    rows1, bas1, rows2, bas2 = _prep(n, e, a0, a1, src2)
    rows1f = rows1.reshape(_S * e)
    bas1f = bas1.reshape(_S * e)
    rows2f = rows2.reshape(_S * e)
    bas2f = bas2.reshape(_S * e)

    z128 = jnp.zeros((n, _D), jnp.float32)

    w1full = jnp.concatenate([conv1_w, conv1_root[None]], axis=0)
    y1 = _proj(x, w1full)
    root1t = lax.slice(y1, (k1 * n, 0), ((k1 + 1) * n, _D))

    (cntp,) = _make_cnt(n, e)(dst, z128)
    (part1,) = _make_agg(n, e)(y1, rows1f, bas1f, dst, z128)

    w2full = jnp.concatenate([conv2_w, conv2_root[None]], axis=0)
    y2 = _fin1proj2(part1, cntp, root1t, conv1_b.reshape(1, _D), w2full)
    root2t = lax.slice(y2, (k2 * n, 0), ((k2 + 1) * n, _D))

    (part2,) = _make_agg(n, e)(y2, rows2f, bas2f, dst, z128)

    out = _fin2mlp(part2, cntp, root2t, conv2_b.reshape(1, _D),
                   mlp1_w, mlp1_b.reshape(1, _D),
                   mlp2_w, mlp2_b.reshape(1, mlp2_b.shape[0]))
    return out


# speedup vs baseline: 7.9768x; 1.2361x over previous
import functools

import jax
import jax.numpy as jnp
from jax import lax
from jax.experimental import pallas as pl
from jax.experimental.pallas import tpu as pltpu
from jax.experimental.pallas import tpu_sc as plsc

_NC = 2
_NS = 16
_NW = _NC * _NS
_L = 16

_D = 128
_S = 4


_B = 80
_IW = 512
_FW = 384


def _prep_body(n, a0_ref, a1_ref, src_ref, dst_ref,
               i1_ref, f1_ref, i2_ref, f2_ref):
    a0 = a0_ref[...]
    a1 = a1_ref[...]
    srcv = src_ref[...]
    dstv = dst_ref[...]
    zi = jnp.zeros((a0.shape[0], _IW - 5 * _B), jnp.int32)
    zf = jnp.zeros((a0.shape[0], _FW - 4 * _B), jnp.float32)
    for ks, i_ref, f_ref in ((3, i1_ref, f1_ref), (5, i2_ref, f2_ref)):
        v0 = a0 * (ks - 1)
        bot0 = jnp.floor(v0)
        f0 = v0 - bot0
        i0 = bot0.astype(jnp.int32)
        v1 = a1 * (ks - 1)
        bot1 = jnp.floor(v1)
        f1 = v1 - bot1
        i1 = bot1.astype(jnp.int32)
        ws, rs = [], []
        for s in range(_S):
            bit0 = s & 1
            bit1 = (s >> 1) & 1
            w0 = f0 if bit0 else 1.0 - f0
            w1 = f1 if bit1 else 1.0 - f1
            idx0 = jnp.clip(i0 + bit0, 0, ks - 1)
            idx1 = jnp.clip(i1 + bit1, 0, ks - 1)
            wi = idx0 + ks * idx1
            ws.append(w0 * w1)
            rs.append(wi * n + srcv)
        i_ref[...] = jnp.concatenate(rs + [dstv, zi], axis=1)
        f_ref[...] = jnp.concatenate(ws + [zf], axis=1)


def _prep(n, e, a0, a1, src2, dst2):
    rows = a0.shape[0]
    grid = 5
    rb = rows // grid
    in_spec = pl.BlockSpec((rb, _B), lambda i: (i, 0))
    f32 = jnp.float32
    return pl.pallas_call(
        functools.partial(_prep_body, n),
        grid=(grid,),
        in_specs=[in_spec, in_spec, in_spec, in_spec],
        out_specs=[pl.BlockSpec((rb, _IW), lambda i: (i, 0)),
                   pl.BlockSpec((rb, _FW), lambda i: (i, 0)),
                   pl.BlockSpec((rb, _IW), lambda i: (i, 0)),
                   pl.BlockSpec((rb, _FW), lambda i: (i, 0))],
        out_shape=[
            jax.ShapeDtypeStruct((rows, _IW), jnp.int32),
            jax.ShapeDtypeStruct((rows, _FW), f32),
            jax.ShapeDtypeStruct((rows, _IW), jnp.int32),
            jax.ShapeDtypeStruct((rows, _FW), f32),
        ],
    )(a0, a1, src2, dst2)


def _proj_body(x_ref, w_ref, o_ref):
    k = pl.program_id(1)
    o_ref[...] = jnp.dot(x_ref[...], w_ref[k],
                         preferred_element_type=jnp.float32)


def _proj(xin, wfull, bn=400):
    n = xin.shape[0]
    k1 = wfull.shape[0]
    nb = n // bn
    return pl.pallas_call(
        _proj_body,
        grid=(nb, k1),
        in_specs=[
            pl.BlockSpec((bn, _D), lambda i, k: (i, 0)),
            pl.BlockSpec((k1, _D, _D), lambda i, k: (0, 0, 0)),
        ],
        out_specs=pl.BlockSpec((bn, _D), lambda i, k: (k * nb + i, 0)),
        out_shape=jax.ShapeDtypeStruct((k1 * n, _D), jnp.float32),
    )(xin, wfull)


def _make_agg(n, e_total):
    epw = e_total // _NW
    B = 80
    nblk = epw // B
    G = B // _L
    chunk = (n // _NS) & ~7
    rem = n - _NS * chunk

    mesh = plsc.VectorSubcoreMesh(core_axis_name="c", subcore_axis_name="s",
                                  num_cores=_NC, num_subcores=_NS)

    out_type = [jax.ShapeDtypeStruct((_NC, n, _D), jnp.float32)]

    scratch = [
        pltpu.VMEM((_IW,), jnp.int32),
        pltpu.VMEM((_FW,), jnp.float32),
        pltpu.VMEM((B,), jnp.int32),
        pltpu.VMEM((B, _D), jnp.float32),
        pltpu.VMEM((B, _D), jnp.float32),
        pltpu.VMEM((B, _D), jnp.float32),
        pltpu.VMEM((B, _D), jnp.float32),
        pltpu.VMEM_SHARED((n, _D), jnp.float32),
        pltpu.SemaphoreType.DMA,
    ]

    def body(table, i_h, f_h, z128, out_p,
             ibuf, fbuf, dstv, t0, t1, t2, t3, acc, sem):
        tbufs = (t0, t1, t2, t3)

        c = lax.axis_index("c")
        s = lax.axis_index("s")
        wid = s * _NC + c

        pltpu.sync_copy(z128.at[pl.ds(s * chunk, chunk)],
                        acc.at[pl.ds(s * chunk, chunk)])
        if rem:
            @pl.when(s == _NS - 1)
            def _zero_rem():
                pltpu.sync_copy(z128.at[pl.ds(_NS * chunk, rem)],
                                acc.at[pl.ds(_NS * chunk, rem)])
        plsc.subcore_barrier()

        base_r = wid * nblk

        def block_body(j, carry):
            row = base_r + j
            pltpu.sync_copy(i_h.at[pl.ds(row * _IW, _IW)], ibuf)
            pltpu.sync_copy(f_h.at[pl.ds(row * _FW, _FW)], fbuf)
            for i in range(B // _L):
                dstv[pl.ds(i * _L, _L)] = ibuf[pl.ds(_S * B + i * _L, _L)]
            cps = [pltpu.async_copy(table.at[ibuf.at[pl.ds(t * B, B)]],
                                    tbufs[t], sem)
                   for t in range(_S)]
            for cp in cps:
                cp.wait()

            def g_body(g, carry2):
                bch = [fbuf[pl.ds(t * B + g * _L, _L)] for t in range(_S)]

                def l_body(l, carry3):
                    e = g * _L + l
                    lidx = jnp.zeros((_L,), jnp.int32) + l
                    bvs = [jnp.take(bch[t], lidx) for t in range(_S)]
                    for cch in range(_D // _L):
                        o = cch * _L
                        accv = None
                        for t in range(_S):
                            v = tbufs[t][e, pl.ds(o, _L)]
                            contrib = v * bvs[t]
                            accv = contrib if accv is None else accv + contrib
                        t0[e, pl.ds(o, _L)] = accv
                    return 0

                lax.fori_loop(0, _L, l_body, 0)
                return 0

            lax.fori_loop(0, G, g_body, 0)

            pltpu.sync_copy(t0, acc.at[dstv], add=True)
            return 0

        lax.fori_loop(0, nblk, block_body, 0)

        plsc.subcore_barrier()
        pltpu.sync_copy(acc.at[pl.ds(s * chunk, chunk)],
                        out_p.at[c, pl.ds(s * chunk, chunk)])
        if rem:
            @pl.when(s == _NS - 1)
            def _out_rem():
                pltpu.sync_copy(acc.at[pl.ds(_NS * chunk, rem)],
                                out_p.at[c, pl.ds(_NS * chunk, rem)])

    return pl.kernel(body, out_type=out_type, mesh=mesh,
                     scratch_types=scratch)


def _make_cnt(n, e_total):
    epw = e_total // _NW
    B = 80
    nblk = epw // B
    chunk = (n // _NS) & ~7
    rem = n - _NS * chunk

    mesh = plsc.VectorSubcoreMesh(core_axis_name="c", subcore_axis_name="s",
                                  num_cores=_NC, num_subcores=_NS)

    def body(dst_h, z16, out_c, dstv, ones, cacc, sem):
        c = lax.axis_index("c")
        s = lax.axis_index("s")
        wid = s * _NC + c

        pltpu.sync_copy(z16.at[pl.ds(s * chunk, chunk)],
                        cacc.at[pl.ds(s * chunk, chunk)])

        def ones_body(r, carry):
            for cch in range(_D // _L):
                ones[r, pl.ds(cch * _L, _L)] = jnp.zeros((_L,), jnp.float32) + 1.0
            return 0

        lax.fori_loop(0, B, ones_body, 0)
        if rem:
            @pl.when(s == _NS - 1)
            def _zero_rem():
                pltpu.sync_copy(z16.at[pl.ds(_NS * chunk, rem)],
                                cacc.at[pl.ds(_NS * chunk, rem)])
        plsc.subcore_barrier()

        base_e = wid * epw

        def block_body(j, carry):
            off = base_e + j * B
            pltpu.sync_copy(dst_h.at[pl.ds(off, B)], dstv)
            pltpu.sync_copy(ones, cacc.at[dstv], add=True)
            return 0

        lax.fori_loop(0, nblk, block_body, 0)

        plsc.subcore_barrier()
        pltpu.sync_copy(cacc.at[pl.ds(s * chunk, chunk)],
                        out_c.at[c, pl.ds(s * chunk, chunk)])
        if rem:
            @pl.when(s == _NS - 1)
            def _out_rem():
                pltpu.sync_copy(cacc.at[pl.ds(_NS * chunk, rem)],
                                out_c.at[c, pl.ds(_NS * chunk, rem)])

    return pl.kernel(
        body,
        out_type=[jax.ShapeDtypeStruct((_NC, n, _D), jnp.float32)],
        mesh=mesh,
        scratch_types=[
            pltpu.VMEM((B,), jnp.int32),
            pltpu.VMEM((B, _D), jnp.float32),
            pltpu.VMEM_SHARED((n, _D), jnp.float32),
            pltpu.SemaphoreType.DMA,
        ])


def _fin1_body(part_ref, cntp_ref, root_ref, b_ref, w_ref, o_ref):
    k = pl.program_id(1)
    aggsum = part_ref[0] + part_ref[1]
    cnt2 = cntp_ref[0] + cntp_ref[1]
    cnt = cnt2[:, 0:1]
    h = aggsum / jnp.maximum(cnt, 1.0) + root_ref[...] + b_ref[0]
    h = jnp.where(h > 0, h, jnp.exp(jnp.minimum(h, 0.0)) - 1.0)
    o_ref[...] = jnp.dot(h, w_ref[k], preferred_element_type=jnp.float32)


def _fin1proj2(part, cntp, root1t, b1, w2full, bn=400):
    n = root1t.shape[0]
    k1 = w2full.shape[0]
    nb = n // bn
    return pl.pallas_call(
        _fin1_body,
        grid=(nb, k1),
        in_specs=[
            pl.BlockSpec((_NC, bn, _D), lambda i, k: (0, i, 0)),
            pl.BlockSpec((_NC, bn, _D), lambda i, k: (0, i, 0)),
            pl.BlockSpec((bn, _D), lambda i, k: (i, 0)),
            pl.BlockSpec((1, _D), lambda i, k: (0, 0)),
            pl.BlockSpec((k1, _D, _D), lambda i, k: (0, 0, 0)),
        ],
        out_specs=pl.BlockSpec((bn, _D), lambda i, k: (k * nb + i, 0)),
        out_shape=jax.ShapeDtypeStruct((k1 * n, _D), jnp.float32),
    )(part, cntp, root1t, b1, w2full)


def _fin2_body(part_ref, cntp_ref, root_ref, b_ref, m1w_ref, m1b_ref,
               m2w_ref, m2b_ref, o_ref):
    aggsum = part_ref[0] + part_ref[1]
    cnt2 = cntp_ref[0] + cntp_ref[1]
    cnt = cnt2[:, 0:1]
    h = aggsum / jnp.maximum(cnt, 1.0) + root_ref[...] + b_ref[0]
    h = jnp.where(h > 0, h, jnp.exp(jnp.minimum(h, 0.0)) - 1.0)
    a = jnp.dot(h, m1w_ref[...], preferred_element_type=jnp.float32)
    a = jnp.maximum(a + m1b_ref[0], 0.0)
    o = jnp.dot(a, m2w_ref[...], preferred_element_type=jnp.float32)
    o_ref[...] = jnp.maximum(o + m2b_ref[0], 0.0)


def _fin2mlp(part, cntp, root2t, b2, m1w, m1b, m2w, m2b, bn=400):
    n = root2t.shape[0]
    co = m2w.shape[1]
    nb = n // bn
    return pl.pallas_call(
        _fin2_body,
        grid=(nb,),
        in_specs=[
            pl.BlockSpec((_NC, bn, _D), lambda i: (0, i, 0)),
            pl.BlockSpec((_NC, bn, _D), lambda i: (0, i, 0)),
            pl.BlockSpec((bn, _D), lambda i: (i, 0)),
            pl.BlockSpec((1, _D), lambda i: (0, 0)),
            pl.BlockSpec((_D, _D), lambda i: (0, 0)),
            pl.BlockSpec((1, _D), lambda i: (0, 0)),
            pl.BlockSpec((_D, co), lambda i: (0, 0)),
            pl.BlockSpec((1, co), lambda i: (0, 0)),
        ],
        out_specs=pl.BlockSpec((bn, co), lambda i: (i, 0)),
        out_shape=jax.ShapeDtypeStruct((n, co), jnp.float32),
    )(part, cntp, root2t, b2, m1w, m1b, m2w, m2b)


def kernel(x, edge_index, edge_attr, conv1_w, conv1_root, conv1_b,
           conv2_w, conv2_root, conv2_b, mlp1_w, mlp1_b, mlp2_w, mlp2_b):
    n = x.shape[0]
    e = edge_index.shape[1]
    k1 = conv1_w.shape[0]
    k2 = conv2_w.shape[0]

    src = edge_index[0]
    dst = edge_index[1]
    rows2d = e // _B
    a0 = edge_attr[:, 0].reshape(rows2d, _B)
    a1 = edge_attr[:, 1].reshape(rows2d, _B)
    src2 = src.reshape(rows2d, _B)
    dst2 = dst.reshape(rows2d, _B)

    i1s, f1s, i2s, f2s = _prep(n, e, a0, a1, src2, dst2)
    i1s = i1s.reshape(rows2d * _IW)
    f1s = f1s.reshape(rows2d * _FW)
    i2s = i2s.reshape(rows2d * _IW)
    f2s = f2s.reshape(rows2d * _FW)

    z128 = jnp.zeros((n, _D), jnp.float32)

    w1full = jnp.concatenate([conv1_w, conv1_root[None]], axis=0)
    y1 = _proj(x, w1full)
    root1t = lax.slice(y1, (k1 * n, 0), ((k1 + 1) * n, _D))

    (cntp,) = _make_cnt(n, e)(dst, z128)
    (part1,) = _make_agg(n, e)(y1, i1s, f1s, z128)

    w2full = jnp.concatenate([conv2_w, conv2_root[None]], axis=0)
    y2 = _fin1proj2(part1, cntp, root1t, conv1_b.reshape(1, _D), w2full)
    root2t = lax.slice(y2, (k2 * n, 0), ((k2 + 1) * n, _D))

    (part2,) = _make_agg(n, e)(y2, i2s, f2s, z128)

    out = _fin2mlp(part2, cntp, root2t, conv2_b.reshape(1, _D),
                   mlp1_w, mlp1_b.reshape(1, _D),
                   mlp2_w, mlp2_b.reshape(1, mlp2_b.shape[0]))
    return out


# speedup vs baseline: 8.6569x; 1.0853x over previous
import functools

import jax
import jax.numpy as jnp
from jax import lax
from jax.experimental import pallas as pl
from jax.experimental.pallas import tpu as pltpu
from jax.experimental.pallas import tpu_sc as plsc

_NC = 2
_NS = 16
_NW = _NC * _NS
_L = 16

_D = 128
_S = 4


_B = 80
_IW = 512
_FW = 384


def _prep_body(n, a0_ref, a1_ref, src_ref, dst_ref,
               i1_ref, f1_ref, i2_ref, f2_ref):
    a0 = a0_ref[...]
    a1 = a1_ref[...]
    srcv = src_ref[...]
    dstv = dst_ref[...]
    zi = jnp.zeros((a0.shape[0], _IW - 5 * _B), jnp.int32)
    zf = jnp.zeros((a0.shape[0], _FW - 4 * _B), jnp.float32)
    for ks, i_ref, f_ref in ((3, i1_ref, f1_ref), (5, i2_ref, f2_ref)):
        v0 = a0 * (ks - 1)
        bot0 = jnp.floor(v0)
        f0 = v0 - bot0
        i0 = bot0.astype(jnp.int32)
        v1 = a1 * (ks - 1)
        bot1 = jnp.floor(v1)
        f1 = v1 - bot1
        i1 = bot1.astype(jnp.int32)
        ws, rs = [], []
        for s in range(_S):
            bit0 = s & 1
            bit1 = (s >> 1) & 1
            w0 = f0 if bit0 else 1.0 - f0
            w1 = f1 if bit1 else 1.0 - f1
            idx0 = jnp.clip(i0 + bit0, 0, ks - 1)
            idx1 = jnp.clip(i1 + bit1, 0, ks - 1)
            wi = idx0 + ks * idx1
            ws.append(w0 * w1)
            rs.append(wi * n + srcv)
        i_ref[...] = jnp.concatenate(rs + [dstv, zi], axis=1)
        f_ref[...] = jnp.concatenate(ws + [zf], axis=1)


def _prep(n, e, a0, a1, src2, dst2):
    rows = a0.shape[0]
    grid = 5
    rb = rows // grid
    in_spec = pl.BlockSpec((rb, _B), lambda i: (i, 0))
    f32 = jnp.float32
    return pl.pallas_call(
        functools.partial(_prep_body, n),
        grid=(grid,),
        in_specs=[in_spec, in_spec, in_spec, in_spec],
        out_specs=[pl.BlockSpec((rb, _IW), lambda i: (i, 0)),
                   pl.BlockSpec((rb, _FW), lambda i: (i, 0)),
                   pl.BlockSpec((rb, _IW), lambda i: (i, 0)),
                   pl.BlockSpec((rb, _FW), lambda i: (i, 0))],
        out_shape=[
            jax.ShapeDtypeStruct((rows, _IW), jnp.int32),
            jax.ShapeDtypeStruct((rows, _FW), f32),
            jax.ShapeDtypeStruct((rows, _IW), jnp.int32),
            jax.ShapeDtypeStruct((rows, _FW), f32),
        ],
    )(a0, a1, src2, dst2)


def _proj_body(x_ref, w_ref, o_ref):
    k = pl.program_id(1)
    o_ref[...] = jnp.dot(x_ref[...], w_ref[k],
                         preferred_element_type=jnp.float32)


def _proj(xin, wfull, bn=400):
    n = xin.shape[0]
    k1 = wfull.shape[0]
    nb = n // bn
    return pl.pallas_call(
        _proj_body,
        grid=(nb, k1),
        in_specs=[
            pl.BlockSpec((bn, _D), lambda i, k: (i, 0)),
            pl.BlockSpec((k1, _D, _D), lambda i, k: (0, 0, 0)),
        ],
        out_specs=pl.BlockSpec((bn, _D), lambda i, k: (k * nb + i, 0)),
        out_shape=jax.ShapeDtypeStruct((k1 * n, _D), jnp.float32),
    )(xin, wfull)


def _make_agg(n, e_total):
    epw = e_total // _NW
    B = 80
    nblk = epw // B
    G = B // _L
    chunk = (n // _NS) & ~7
    rem = n - _NS * chunk

    mesh = plsc.VectorSubcoreMesh(core_axis_name="c", subcore_axis_name="s",
                                  num_cores=_NC, num_subcores=_NS)

    out_type = [jax.ShapeDtypeStruct((_NC, n, _D), jnp.float32)]

    H1 = 48
    G1 = H1 // _L

    scratch = [
        pltpu.VMEM((_IW,), jnp.int32),
        pltpu.VMEM((_FW,), jnp.float32),
        pltpu.VMEM((_IW,), jnp.int32),
        pltpu.VMEM((_FW,), jnp.float32),
        pltpu.VMEM((B,), jnp.int32),
        pltpu.VMEM((B, _D), jnp.float32),
        pltpu.VMEM((B, _D), jnp.float32),
        pltpu.VMEM((B, _D), jnp.float32),
        pltpu.VMEM((B, _D), jnp.float32),
        pltpu.VMEM_SHARED((n, _D), jnp.float32),
        pltpu.SemaphoreType.DMA,
        pltpu.SemaphoreType.DMA,
        pltpu.SemaphoreType.DMA,
    ]

    def body(table, i_h, f_h, z128, out_p,
             ibufa, fbufa, ibufb, fbufb, dstv, t0, t1, t2, t3, acc,
             sem1, sem2, sems):
        tbufs = (t0, t1, t2, t3)

        c = lax.axis_index("c")
        s = lax.axis_index("s")
        wid = s * _NC + c

        pltpu.sync_copy(z128.at[pl.ds(s * chunk, chunk)],
                        acc.at[pl.ds(s * chunk, chunk)])
        if rem:
            @pl.when(s == _NS - 1)
            def _zero_rem():
                pltpu.sync_copy(z128.at[pl.ds(_NS * chunk, rem)],
                                acc.at[pl.ds(_NS * chunk, rem)])
        plsc.subcore_barrier()

        base_r = wid * nblk

        def compute_groups(fbuf, g_lo, g_hi):
            def g_body(g, carry2):
                bch = [fbuf[pl.ds(t * B + g * _L, _L)] for t in range(_S)]

                def l_body(l, carry3):
                    e = g * _L + l
                    lidx = jnp.zeros((_L,), jnp.int32) + l
                    bvs = [jnp.take(bch[t], lidx) for t in range(_S)]
                    for cch in range(_D // _L):
                        o = cch * _L
                        accv = None
                        for t in range(_S):
                            v = tbufs[t][e, pl.ds(o, _L)]
                            contrib = v * bvs[t]
                            accv = contrib if accv is None else accv + contrib
                        t0[e, pl.ds(o, _L)] = accv
                    return 0

                lax.fori_loop(0, _L, l_body, 0)
                return 0

            lax.fori_loop(g_lo, g_hi, g_body, 0)

        def do_block(j, ibuf, fbuf, inext, fnext, prefetch):
            for i in range(B // _L):
                dstv[pl.ds(i * _L, _L)] = ibuf[pl.ds(_S * B + i * _L, _L)]
            cps1 = [pltpu.async_copy(table.at[ibuf.at[pl.ds(t * B, H1)]],
                                     tbufs[t].at[pl.ds(0, H1)], sem1)
                    for t in range(_S)]
            cps2 = [pltpu.async_copy(
                        table.at[ibuf.at[pl.ds(t * B + H1, B - H1)]],
                        tbufs[t].at[pl.ds(H1, B - H1)], sem2)
                    for t in range(_S)]
            if prefetch:
                row2 = base_r + j + 1
                cpi = pltpu.async_copy(i_h.at[pl.ds(row2 * _IW, _IW)],
                                       inext, sems)
                cpf = pltpu.async_copy(f_h.at[pl.ds(row2 * _FW, _FW)],
                                       fnext, sems)
            for cp in cps1:
                cp.wait()
            compute_groups(fbuf, 0, G1)
            for cp in cps2:
                cp.wait()
            compute_groups(fbuf, G1, G)
            pltpu.sync_copy(t0, acc.at[dstv], add=True)
            if prefetch:
                cpi.wait()
                cpf.wait()

        pltpu.sync_copy(i_h.at[pl.ds(base_r * _IW, _IW)], ibufa)
        pltpu.sync_copy(f_h.at[pl.ds(base_r * _FW, _FW)], fbufa)

        def pair_body(j2, carry):
            j = j2 * 2
            do_block(j, ibufa, fbufa, ibufb, fbufb, True)
            do_block(j + 1, ibufb, fbufb, ibufa, fbufa, True)
            return 0

        lax.fori_loop(0, (nblk - 1) // 2, pair_body, 0)
        do_block(nblk - 1, ibufa, fbufa, ibufb, fbufb, False)

        plsc.subcore_barrier()
        pltpu.sync_copy(acc.at[pl.ds(s * chunk, chunk)],
                        out_p.at[c, pl.ds(s * chunk, chunk)])
        if rem:
            @pl.when(s == _NS - 1)
            def _out_rem():
                pltpu.sync_copy(acc.at[pl.ds(_NS * chunk, rem)],
                                out_p.at[c, pl.ds(_NS * chunk, rem)])

    return pl.kernel(body, out_type=out_type, mesh=mesh,
                     scratch_types=scratch)


def _make_cnt(n, e_total):
    epw = e_total // _NW
    B = 80
    nblk = epw // B
    chunk = (n // _NS) & ~7
    rem = n - _NS * chunk

    mesh = plsc.VectorSubcoreMesh(core_axis_name="c", subcore_axis_name="s",
                                  num_cores=_NC, num_subcores=_NS)

    def body(dst_h, z16, out_c, dstv, ones, cacc, sem):
        c = lax.axis_index("c")
        s = lax.axis_index("s")
        wid = s * _NC + c

        pltpu.sync_copy(z16.at[pl.ds(s * chunk, chunk)],
                        cacc.at[pl.ds(s * chunk, chunk)])

        def ones_body(r, carry):
            for cch in range(_D // _L):
                ones[r, pl.ds(cch * _L, _L)] = jnp.zeros((_L,), jnp.float32) + 1.0
            return 0

        lax.fori_loop(0, B, ones_body, 0)
        if rem:
            @pl.when(s == _NS - 1)
            def _zero_rem():
                pltpu.sync_copy(z16.at[pl.ds(_NS * chunk, rem)],
                                cacc.at[pl.ds(_NS * chunk, rem)])
        plsc.subcore_barrier()

        base_e = wid * epw

        def block_body(j, carry):
            off = base_e + j * B
            pltpu.sync_copy(dst_h.at[pl.ds(off, B)], dstv)
            pltpu.sync_copy(ones, cacc.at[dstv], add=True)
            return 0

        lax.fori_loop(0, nblk, block_body, 0)

        plsc.subcore_barrier()
        pltpu.sync_copy(cacc.at[pl.ds(s * chunk, chunk)],
                        out_c.at[c, pl.ds(s * chunk, chunk)])
        if rem:
            @pl.when(s == _NS - 1)
            def _out_rem():
                pltpu.sync_copy(cacc.at[pl.ds(_NS * chunk, rem)],
                                out_c.at[c, pl.ds(_NS * chunk, rem)])

    return pl.kernel(
        body,
        out_type=[jax.ShapeDtypeStruct((_NC, n, _D), jnp.float32)],
        mesh=mesh,
        scratch_types=[
            pltpu.VMEM((B,), jnp.int32),
            pltpu.VMEM((B, _D), jnp.float32),
            pltpu.VMEM_SHARED((n, _D), jnp.float32),
            pltpu.SemaphoreType.DMA,
        ])


def _fin1_body(part_ref, cntp_ref, root_ref, b_ref, w_ref, o_ref):
    k = pl.program_id(1)
    aggsum = part_ref[0] + part_ref[1]
    cnt2 = cntp_ref[0] + cntp_ref[1]
    cnt = cnt2[:, 0:1]
    h = aggsum / jnp.maximum(cnt, 1.0) + root_ref[...] + b_ref[0]
    h = jnp.where(h > 0, h, jnp.exp(jnp.minimum(h, 0.0)) - 1.0)
    o_ref[...] = jnp.dot(h, w_ref[k], preferred_element_type=jnp.float32)


def _fin1proj2(part, cntp, root1t, b1, w2full, bn=400):
    n = root1t.shape[0]
    k1 = w2full.shape[0]
    nb = n // bn
    return pl.pallas_call(
        _fin1_body,
        grid=(nb, k1),
        in_specs=[
            pl.BlockSpec((_NC, bn, _D), lambda i, k: (0, i, 0)),
            pl.BlockSpec((_NC, bn, _D), lambda i, k: (0, i, 0)),
            pl.BlockSpec((bn, _D), lambda i, k: (i, 0)),
            pl.BlockSpec((1, _D), lambda i, k: (0, 0)),
            pl.BlockSpec((k1, _D, _D), lambda i, k: (0, 0, 0)),
        ],
        out_specs=pl.BlockSpec((bn, _D), lambda i, k: (k * nb + i, 0)),
        out_shape=jax.ShapeDtypeStruct((k1 * n, _D), jnp.float32),
    )(part, cntp, root1t, b1, w2full)


def _fin2_body(part_ref, cntp_ref, root_ref, b_ref, m1w_ref, m1b_ref,
               m2w_ref, m2b_ref, o_ref):
    aggsum = part_ref[0] + part_ref[1]
    cnt2 = cntp_ref[0] + cntp_ref[1]
    cnt = cnt2[:, 0:1]
    h = aggsum / jnp.maximum(cnt, 1.0) + root_ref[...] + b_ref[0]
    h = jnp.where(h > 0, h, jnp.exp(jnp.minimum(h, 0.0)) - 1.0)
    a = jnp.dot(h, m1w_ref[...], preferred_element_type=jnp.float32)
    a = jnp.maximum(a + m1b_ref[0], 0.0)
    o = jnp.dot(a, m2w_ref[...], preferred_element_type=jnp.float32)
    o_ref[...] = jnp.maximum(o + m2b_ref[0], 0.0)


def _fin2mlp(part, cntp, root2t, b2, m1w, m1b, m2w, m2b, bn=400):
    n = root2t.shape[0]
    co = m2w.shape[1]
    nb = n // bn
    return pl.pallas_call(
        _fin2_body,
        grid=(nb,),
        in_specs=[
            pl.BlockSpec((_NC, bn, _D), lambda i: (0, i, 0)),
            pl.BlockSpec((_NC, bn, _D), lambda i: (0, i, 0)),
            pl.BlockSpec((bn, _D), lambda i: (i, 0)),
            pl.BlockSpec((1, _D), lambda i: (0, 0)),
            pl.BlockSpec((_D, _D), lambda i: (0, 0)),
            pl.BlockSpec((1, _D), lambda i: (0, 0)),
            pl.BlockSpec((_D, co), lambda i: (0, 0)),
            pl.BlockSpec((1, co), lambda i: (0, 0)),
        ],
        out_specs=pl.BlockSpec((bn, co), lambda i: (i, 0)),
        out_shape=jax.ShapeDtypeStruct((n, co), jnp.float32),
    )(part, cntp, root2t, b2, m1w, m1b, m2w, m2b)


def kernel(x, edge_index, edge_attr, conv1_w, conv1_root, conv1_b,
           conv2_w, conv2_root, conv2_b, mlp1_w, mlp1_b, mlp2_w, mlp2_b):
    n = x.shape[0]
    e = edge_index.shape[1]
    k1 = conv1_w.shape[0]
    k2 = conv2_w.shape[0]

    src = edge_index[0]
    dst = edge_index[1]
    rows2d = e // _B
    a0 = edge_attr[:, 0].reshape(rows2d, _B)
    a1 = edge_attr[:, 1].reshape(rows2d, _B)
    src2 = src.reshape(rows2d, _B)
    dst2 = dst.reshape(rows2d, _B)

    i1s, f1s, i2s, f2s = _prep(n, e, a0, a1, src2, dst2)
    i1s = i1s.reshape(rows2d * _IW)
    f1s = f1s.reshape(rows2d * _FW)
    i2s = i2s.reshape(rows2d * _IW)
    f2s = f2s.reshape(rows2d * _FW)

    z128 = jnp.zeros((n, _D), jnp.float32)

    w1full = jnp.concatenate([conv1_w, conv1_root[None]], axis=0)
    y1 = _proj(x, w1full)
    root1t = lax.slice(y1, (k1 * n, 0), ((k1 + 1) * n, _D))

    (cntp,) = _make_cnt(n, e)(dst, z128)
    (part1,) = _make_agg(n, e)(y1, i1s, f1s, z128)

    w2full = jnp.concatenate([conv2_w, conv2_root[None]], axis=0)
    y2 = _fin1proj2(part1, cntp, root1t, conv1_b.reshape(1, _D), w2full)
    root2t = lax.slice(y2, (k2 * n, 0), ((k2 + 1) * n, _D))

    (part2,) = _make_agg(n, e)(y2, i2s, f2s, z128)

    out = _fin2mlp(part2, cntp, root2t, conv2_b.reshape(1, _D),
                   mlp1_w, mlp1_b.reshape(1, _D),
                   mlp2_w, mlp2_b.reshape(1, mlp2_b.shape[0]))
    return out


# speedup vs baseline: 11.1284x; 1.2855x over previous
import functools

import jax
import jax.numpy as jnp
from jax import lax
from jax.experimental import pallas as pl
from jax.experimental.pallas import tpu as pltpu
from jax.experimental.pallas import tpu_sc as plsc

_NC = 2
_NS = 16
_NW = _NC * _NS
_L = 16

_D = 128
_S = 4


_B = 80
_IW = 512
_FW = 384


def _prep_body(n, a0_ref, a1_ref, src_ref, dst_ref,
               i1_ref, f1_ref, i2_ref, f2_ref):
    a0 = a0_ref[...]
    a1 = a1_ref[...]
    srcv = src_ref[...]
    dstv = dst_ref[...]
    zi = jnp.zeros((a0.shape[0], _IW - 5 * _B), jnp.int32)
    zf = jnp.zeros((a0.shape[0], _FW - 4 * _B), jnp.float32)
    for ks, i_ref, f_ref in ((3, i1_ref, f1_ref), (5, i2_ref, f2_ref)):
        v0 = a0 * (ks - 1)
        bot0 = jnp.floor(v0)
        f0 = v0 - bot0
        i0 = bot0.astype(jnp.int32)
        v1 = a1 * (ks - 1)
        bot1 = jnp.floor(v1)
        f1 = v1 - bot1
        i1 = bot1.astype(jnp.int32)
        ws, rs = [], []
        for s in range(_S):
            bit0 = s & 1
            bit1 = (s >> 1) & 1
            w0 = f0 if bit0 else 1.0 - f0
            w1 = f1 if bit1 else 1.0 - f1
            idx0 = jnp.clip(i0 + bit0, 0, ks - 1)
            idx1 = jnp.clip(i1 + bit1, 0, ks - 1)
            wi = idx0 + ks * idx1
            ws.append(w0 * w1)
            rs.append(wi * n + srcv)
        i_ref[...] = jnp.concatenate(rs + [dstv, zi], axis=1)
        f_ref[...] = jnp.concatenate(ws + [zf], axis=1)


def _prep(n, e, a0, a1, src2, dst2):
    rows = a0.shape[0]
    grid = 5
    rb = rows // grid
    in_spec = pl.BlockSpec((rb, _B), lambda i: (i, 0))
    f32 = jnp.float32
    return pl.pallas_call(
        functools.partial(_prep_body, n),
        grid=(grid,),
        in_specs=[in_spec, in_spec, in_spec, in_spec],
        out_specs=[pl.BlockSpec((rb, _IW), lambda i: (i, 0)),
                   pl.BlockSpec((rb, _FW), lambda i: (i, 0)),
                   pl.BlockSpec((rb, _IW), lambda i: (i, 0)),
                   pl.BlockSpec((rb, _FW), lambda i: (i, 0))],
        out_shape=[
            jax.ShapeDtypeStruct((rows, _IW), jnp.int32),
            jax.ShapeDtypeStruct((rows, _FW), f32),
            jax.ShapeDtypeStruct((rows, _IW), jnp.int32),
            jax.ShapeDtypeStruct((rows, _FW), f32),
        ],
    )(a0, a1, src2, dst2)


def _proj_body(x_ref, w_ref, o_ref):
    k = pl.program_id(1)
    o_ref[...] = jnp.dot(x_ref[...], w_ref[k],
                         preferred_element_type=jnp.float32)


def _proj(xin, wfull, bn=400):
    n = xin.shape[0]
    k1 = wfull.shape[0]
    nb = n // bn
    return pl.pallas_call(
        _proj_body,
        grid=(nb, k1),
        in_specs=[
            pl.BlockSpec((bn, _D), lambda i, k: (i, 0)),
            pl.BlockSpec((k1, _D, _D), lambda i, k: (0, 0, 0)),
        ],
        out_specs=pl.BlockSpec((bn, _D), lambda i, k: (k * nb + i, 0)),
        out_shape=jax.ShapeDtypeStruct((k1 * n, _D), jnp.float32),
    )(xin, wfull)


def _make_agg(n, e_total):
    epw = e_total // _NW
    B = 80
    nblk = epw // B
    G = B // _L
    chunk = (n // _NS) & ~7
    rem = n - _NS * chunk

    mesh = plsc.VectorSubcoreMesh(core_axis_name="c", subcore_axis_name="s",
                                  num_cores=_NC, num_subcores=_NS)

    out_type = [jax.ShapeDtypeStruct((_NC, n, _D), jnp.float32)]

    H1 = 48
    G1 = H1 // _L

    scratch = [
        pltpu.VMEM((_IW,), jnp.int32),
        pltpu.VMEM((_FW,), jnp.float32),
        pltpu.VMEM((_IW,), jnp.int32),
        pltpu.VMEM((_FW,), jnp.float32),
        pltpu.VMEM((H1,), jnp.int32),
        pltpu.VMEM((B - H1,), jnp.int32),
        pltpu.VMEM((B, _D), jnp.float32),
        pltpu.VMEM((B, _D), jnp.float32),
        pltpu.VMEM((B, _D), jnp.float32),
        pltpu.VMEM((B, _D), jnp.float32),
        pltpu.VMEM_SHARED((n, _D), jnp.float32),
        pltpu.SemaphoreType.DMA,
        pltpu.SemaphoreType.DMA,
        pltpu.SemaphoreType.DMA,
    ]

    def body(table, i_h, f_h, z128, out_p,
             ibufa, fbufa, ibufb, fbufb, dstv1, dstv2, t0, t1, t2, t3, acc,
             sem1, sem2, sems):
        tbufs = (t0, t1, t2, t3)

        c = lax.axis_index("c")
        s = lax.axis_index("s")
        wid = s * _NC + c

        pltpu.sync_copy(z128.at[pl.ds(s * chunk, chunk)],
                        acc.at[pl.ds(s * chunk, chunk)])
        if rem:
            @pl.when(s == _NS - 1)
            def _zero_rem():
                pltpu.sync_copy(z128.at[pl.ds(_NS * chunk, rem)],
                                acc.at[pl.ds(_NS * chunk, rem)])
        plsc.subcore_barrier()

        base_r = wid * nblk

        def compute_groups(fbuf, g_lo, g_hi):
            def g_body(g, carry2):
                bch = [fbuf[pl.ds(t * B + g * _L, _L)] for t in range(_S)]

                def l_body(l, carry3):
                    e = g * _L + l
                    lidx = jnp.zeros((_L,), jnp.int32) + l
                    bvs = [jnp.take(bch[t], lidx) for t in range(_S)]
                    for cch in range(_D // _L):
                        o = cch * _L
                        accv = None
                        for t in range(_S):
                            v = tbufs[t][e, pl.ds(o, _L)]
                            contrib = v * bvs[t]
                            accv = contrib if accv is None else accv + contrib
                        t0[e, pl.ds(o, _L)] = accv
                    return 0

                lax.fori_loop(0, _L, l_body, 0, unroll=4)
                return 0

            lax.fori_loop(g_lo, g_hi, g_body, 0)

        def extract_dst(ibuf):
            for i in range(G1):
                dstv1[pl.ds(i * _L, _L)] = ibuf[pl.ds(_S * B + i * _L, _L)]
            for i in range(G - G1):
                dstv2[pl.ds(i * _L, _L)] = ibuf[
                    pl.ds(_S * B + H1 + i * _L, _L)]

        def issue_half1(ibuf):
            return [pltpu.async_copy(table.at[ibuf.at[pl.ds(t * B, H1)]],
                                     tbufs[t].at[pl.ds(0, H1)], sem1)
                    for t in range(_S)]

        def issue_half2(ibuf):
            return [pltpu.async_copy(
                        table.at[ibuf.at[pl.ds(t * B + H1, B - H1)]],
                        tbufs[t].at[pl.ds(H1, B - H1)], sem2)
                    for t in range(_S)]

        def stage(j, ibuf, fbuf, inext, fnext, pipelined):
            if pipelined:
                row2 = base_r + j + 1
                cpi = pltpu.async_copy(i_h.at[pl.ds(row2 * _IW, _IW)],
                                       inext, sems)
                cpf = pltpu.async_copy(f_h.at[pl.ds(row2 * _FW, _FW)],
                                       fnext, sems)
            for cp in _cps1[0]:
                cp.wait()
            compute_groups(fbuf, 0, G1)
            pltpu.sync_copy(t0.at[pl.ds(0, H1)], acc.at[dstv1], add=True)
            if pipelined:
                cpi.wait()
                cpf.wait()
            for cp in _cps2[0]:
                cp.wait()
            if pipelined:
                _cps1[0] = issue_half1(inext)
            compute_groups(fbuf, G1, G)
            pltpu.sync_copy(t0.at[pl.ds(H1, B - H1)], acc.at[dstv2],
                            add=True)
            if pipelined:
                extract_dst(inext)
                _cps2[0] = issue_half2(inext)

        pltpu.sync_copy(i_h.at[pl.ds(base_r * _IW, _IW)], ibufa)
        pltpu.sync_copy(f_h.at[pl.ds(base_r * _FW, _FW)], fbufa)
        extract_dst(ibufa)
        _cps1 = [issue_half1(ibufa)]
        _cps2 = [issue_half2(ibufa)]

        def pair_body(j2, carry):
            j = j2 * 2
            stage(j, ibufa, fbufa, ibufb, fbufb, True)
            stage(j + 1, ibufb, fbufb, ibufa, fbufa, True)
            return 0

        lax.fori_loop(0, (nblk - 1) // 2, pair_body, 0)
        stage(nblk - 1, ibufa, fbufa, ibufb, fbufb, False)

        plsc.subcore_barrier()
        pltpu.sync_copy(acc.at[pl.ds(s * chunk, chunk)],
                        out_p.at[c, pl.ds(s * chunk, chunk)])
        if rem:
            @pl.when(s == _NS - 1)
            def _out_rem():
                pltpu.sync_copy(acc.at[pl.ds(_NS * chunk, rem)],
                                out_p.at[c, pl.ds(_NS * chunk, rem)])

    return pl.kernel(body, out_type=out_type, mesh=mesh,
                     scratch_types=scratch)


def _make_cnt(n, e_total):
    epw = e_total // _NW
    B = 80
    nblk = epw // B
    chunk = (n // _NS) & ~7
    rem = n - _NS * chunk

    mesh = plsc.VectorSubcoreMesh(core_axis_name="c", subcore_axis_name="s",
                                  num_cores=_NC, num_subcores=_NS)

    def body(dst_h, z16, out_c, dstv, ones, cacc, sem):
        c = lax.axis_index("c")
        s = lax.axis_index("s")
        wid = s * _NC + c

        pltpu.sync_copy(z16.at[pl.ds(s * chunk, chunk)],
                        cacc.at[pl.ds(s * chunk, chunk)])

        def ones_body(r, carry):
            for cch in range(_D // _L):
                ones[r, pl.ds(cch * _L, _L)] = jnp.zeros((_L,), jnp.float32) + 1.0
            return 0

        lax.fori_loop(0, B, ones_body, 0)
        if rem:
            @pl.when(s == _NS - 1)
            def _zero_rem():
                pltpu.sync_copy(z16.at[pl.ds(_NS * chunk, rem)],
                                cacc.at[pl.ds(_NS * chunk, rem)])
        plsc.subcore_barrier()

        base_e = wid * epw

        def block_body(j, carry):
            off = base_e + j * B
            pltpu.sync_copy(dst_h.at[pl.ds(off, B)], dstv)
            pltpu.sync_copy(ones, cacc.at[dstv], add=True)
            return 0

        lax.fori_loop(0, nblk, block_body, 0)

        plsc.subcore_barrier()
        pltpu.sync_copy(cacc.at[pl.ds(s * chunk, chunk)],
                        out_c.at[c, pl.ds(s * chunk, chunk)])
        if rem:
            @pl.when(s == _NS - 1)
            def _out_rem():
                pltpu.sync_copy(cacc.at[pl.ds(_NS * chunk, rem)],
                                out_c.at[c, pl.ds(_NS * chunk, rem)])

    return pl.kernel(
        body,
        out_type=[jax.ShapeDtypeStruct((_NC, n, _D), jnp.float32)],
        mesh=mesh,
        scratch_types=[
            pltpu.VMEM((B,), jnp.int32),
            pltpu.VMEM((B, _D), jnp.float32),
            pltpu.VMEM_SHARED((n, _D), jnp.float32),
            pltpu.SemaphoreType.DMA,
        ])


def _fin1_body(part_ref, cntp_ref, root_ref, b_ref, w_ref, o_ref):
    k = pl.program_id(1)
    aggsum = part_ref[0] + part_ref[1]
    cnt2 = cntp_ref[0] + cntp_ref[1]
    cnt = cnt2[:, 0:1]
    h = aggsum / jnp.maximum(cnt, 1.0) + root_ref[...] + b_ref[0]
    h = jnp.where(h > 0, h, jnp.exp(jnp.minimum(h, 0.0)) - 1.0)
    o_ref[...] = jnp.dot(h, w_ref[k], preferred_element_type=jnp.float32)


def _fin1proj2(part, cntp, root1t, b1, w2full, bn=400):
    n = root1t.shape[0]
    k1 = w2full.shape[0]
    nb = n // bn
    return pl.pallas_call(
        _fin1_body,
        grid=(nb, k1),
        in_specs=[
            pl.BlockSpec((_NC, bn, _D), lambda i, k: (0, i, 0)),
            pl.BlockSpec((_NC, bn, _D), lambda i, k: (0, i, 0)),
            pl.BlockSpec((bn, _D), lambda i, k: (i, 0)),
            pl.BlockSpec((1, _D), lambda i, k: (0, 0)),
            pl.BlockSpec((k1, _D, _D), lambda i, k: (0, 0, 0)),
        ],
        out_specs=pl.BlockSpec((bn, _D), lambda i, k: (k * nb + i, 0)),
        out_shape=jax.ShapeDtypeStruct((k1 * n, _D), jnp.float32),
    )(part, cntp, root1t, b1, w2full)


def _fin2_body(part_ref, cntp_ref, root_ref, b_ref, m1w_ref, m1b_ref,
               m2w_ref, m2b_ref, o_ref):
    aggsum = part_ref[0] + part_ref[1]
    cnt2 = cntp_ref[0] + cntp_ref[1]
    cnt = cnt2[:, 0:1]
    h = aggsum / jnp.maximum(cnt, 1.0) + root_ref[...] + b_ref[0]
    h = jnp.where(h > 0, h, jnp.exp(jnp.minimum(h, 0.0)) - 1.0)
    a = jnp.dot(h, m1w_ref[...], preferred_element_type=jnp.float32)
    a = jnp.maximum(a + m1b_ref[0], 0.0)
    o = jnp.dot(a, m2w_ref[...], preferred_element_type=jnp.float32)
    o_ref[...] = jnp.maximum(o + m2b_ref[0], 0.0)


def _fin2mlp(part, cntp, root2t, b2, m1w, m1b, m2w, m2b, bn=400):
    n = root2t.shape[0]
    co = m2w.shape[1]
    nb = n // bn
    return pl.pallas_call(
        _fin2_body,
        grid=(nb,),
        in_specs=[
            pl.BlockSpec((_NC, bn, _D), lambda i: (0, i, 0)),
            pl.BlockSpec((_NC, bn, _D), lambda i: (0, i, 0)),
            pl.BlockSpec((bn, _D), lambda i: (i, 0)),
            pl.BlockSpec((1, _D), lambda i: (0, 0)),
            pl.BlockSpec((_D, _D), lambda i: (0, 0)),
            pl.BlockSpec((1, _D), lambda i: (0, 0)),
            pl.BlockSpec((_D, co), lambda i: (0, 0)),
            pl.BlockSpec((1, co), lambda i: (0, 0)),
        ],
        out_specs=pl.BlockSpec((bn, co), lambda i: (i, 0)),
        out_shape=jax.ShapeDtypeStruct((n, co), jnp.float32),
    )(part, cntp, root2t, b2, m1w, m1b, m2w, m2b)


def kernel(x, edge_index, edge_attr, conv1_w, conv1_root, conv1_b,
           conv2_w, conv2_root, conv2_b, mlp1_w, mlp1_b, mlp2_w, mlp2_b):
    n = x.shape[0]
    e = edge_index.shape[1]
    k1 = conv1_w.shape[0]
    k2 = conv2_w.shape[0]

    src = edge_index[0]
    dst = edge_index[1]
    rows2d = e // _B
    a0 = edge_attr[:, 0].reshape(rows2d, _B)
    a1 = edge_attr[:, 1].reshape(rows2d, _B)
    src2 = src.reshape(rows2d, _B)
    dst2 = dst.reshape(rows2d, _B)

    i1s, f1s, i2s, f2s = _prep(n, e, a0, a1, src2, dst2)
    i1s = i1s.reshape(rows2d * _IW)
    f1s = f1s.reshape(rows2d * _FW)
    i2s = i2s.reshape(rows2d * _IW)
    f2s = f2s.reshape(rows2d * _FW)

    z128 = jnp.zeros((n, _D), jnp.float32)

    w1full = jnp.concatenate([conv1_w, conv1_root[None]], axis=0)
    y1 = _proj(x, w1full)
    root1t = lax.slice(y1, (k1 * n, 0), ((k1 + 1) * n, _D))

    (cntp,) = _make_cnt(n, e)(dst, z128)
    (part1,) = _make_agg(n, e)(y1, i1s, f1s, z128)

    w2full = jnp.concatenate([conv2_w, conv2_root[None]], axis=0)
    y2 = _fin1proj2(part1, cntp, root1t, conv1_b.reshape(1, _D), w2full)
    root2t = lax.slice(y2, (k2 * n, 0), ((k2 + 1) * n, _D))

    (part2,) = _make_agg(n, e)(y2, i2s, f2s, z128)

    out = _fin2mlp(part2, cntp, root2t, conv2_b.reshape(1, _D),
                   mlp1_w, mlp1_b.reshape(1, _D),
                   mlp2_w, mlp2_b.reshape(1, mlp2_b.shape[0]))
    return out


# speedup vs baseline: 11.3567x; 1.0205x over previous
import functools

import jax
import jax.numpy as jnp
from jax import lax
from jax.experimental import pallas as pl
from jax.experimental.pallas import tpu as pltpu
from jax.experimental.pallas import tpu_sc as plsc

_NC = 2
_NS = 16
_NW = _NC * _NS
_L = 16

_D = 128
_S = 4


_B = 80
_IW = 512
_FW = 384


def _prep_body(n, a0_ref, a1_ref, src_ref, dst_ref,
               i1_ref, f1_ref, i2_ref, f2_ref):
    a0 = a0_ref[...]
    a1 = a1_ref[...]
    srcv = src_ref[...]
    dstv = dst_ref[...]
    zi = jnp.zeros((a0.shape[0], _IW - 5 * _B), jnp.int32)
    zf = jnp.zeros((a0.shape[0], _FW - 4 * _B), jnp.float32)
    for ks, i_ref, f_ref in ((3, i1_ref, f1_ref), (5, i2_ref, f2_ref)):
        v0 = a0 * (ks - 1)
        bot0 = jnp.floor(v0)
        f0 = v0 - bot0
        i0 = bot0.astype(jnp.int32)
        v1 = a1 * (ks - 1)
        bot1 = jnp.floor(v1)
        f1 = v1 - bot1
        i1 = bot1.astype(jnp.int32)
        ws, rs = [], []
        for s in range(_S):
            bit0 = s & 1
            bit1 = (s >> 1) & 1
            w0 = f0 if bit0 else 1.0 - f0
            w1 = f1 if bit1 else 1.0 - f1
            idx0 = jnp.clip(i0 + bit0, 0, ks - 1)
            idx1 = jnp.clip(i1 + bit1, 0, ks - 1)
            wi = idx0 + ks * idx1
            ws.append(w0 * w1)
            rs.append(wi * n + srcv)
        i_ref[...] = jnp.concatenate(rs + [dstv, zi], axis=1)
        f_ref[...] = jnp.concatenate(ws + [zf], axis=1)


def _prep(n, e, a0, a1, src2, dst2):
    rows = a0.shape[0]
    grid = 5
    rb = rows // grid
    in_spec = pl.BlockSpec((rb, _B), lambda i: (i, 0))
    f32 = jnp.float32
    return pl.pallas_call(
        functools.partial(_prep_body, n),
        grid=(grid,),
        in_specs=[in_spec, in_spec, in_spec, in_spec],
        out_specs=[pl.BlockSpec((rb, _IW), lambda i: (i, 0)),
                   pl.BlockSpec((rb, _FW), lambda i: (i, 0)),
                   pl.BlockSpec((rb, _IW), lambda i: (i, 0)),
                   pl.BlockSpec((rb, _FW), lambda i: (i, 0))],
        out_shape=[
            jax.ShapeDtypeStruct((rows, _IW), jnp.int32),
            jax.ShapeDtypeStruct((rows, _FW), f32),
            jax.ShapeDtypeStruct((rows, _IW), jnp.int32),
            jax.ShapeDtypeStruct((rows, _FW), f32),
        ],
    )(a0, a1, src2, dst2)


def _proj_body(x_ref, w_ref, o_ref):
    k = pl.program_id(1)
    o_ref[...] = jnp.dot(x_ref[...], w_ref[k],
                         preferred_element_type=jnp.float32)


def _proj(xin, wfull, bn=400):
    n = xin.shape[0]
    k1 = wfull.shape[0]
    nb = n // bn
    return pl.pallas_call(
        _proj_body,
        grid=(nb, k1),
        in_specs=[
            pl.BlockSpec((bn, _D), lambda i, k: (i, 0)),
            pl.BlockSpec((k1, _D, _D), lambda i, k: (0, 0, 0)),
        ],
        out_specs=pl.BlockSpec((bn, _D), lambda i, k: (k * nb + i, 0)),
        out_shape=jax.ShapeDtypeStruct((k1 * n, _D), jnp.float32),
    )(xin, wfull)


def _make_agg(n, e_total):
    epw = e_total // _NW
    B = 80
    nblk = epw // B
    G = B // _L
    chunk = (n // _NS) & ~7
    rem = n - _NS * chunk

    mesh = plsc.VectorSubcoreMesh(core_axis_name="c", subcore_axis_name="s",
                                  num_cores=_NC, num_subcores=_NS)

    out_type = [jax.ShapeDtypeStruct((_NC, n, _D), jnp.float32)]

    H1 = 48
    G1 = H1 // _L

    scratch = [
        pltpu.VMEM((_IW,), jnp.int32),
        pltpu.VMEM((_FW,), jnp.float32),
        pltpu.VMEM((_IW,), jnp.int32),
        pltpu.VMEM((_FW,), jnp.float32),
        pltpu.VMEM((H1,), jnp.int32),
        pltpu.VMEM((B - H1,), jnp.int32),
        pltpu.VMEM((H1,), jnp.int32),
        pltpu.VMEM((B - H1,), jnp.int32),
        pltpu.VMEM((B, _D), jnp.float32),
        pltpu.VMEM((B, _D), jnp.float32),
        pltpu.VMEM((B, _D), jnp.float32),
        pltpu.VMEM((B, _D), jnp.float32),
        pltpu.VMEM_SHARED((n, _D), jnp.float32),
        pltpu.SemaphoreType.DMA,
        pltpu.SemaphoreType.DMA,
        pltpu.SemaphoreType.DMA,
        pltpu.SemaphoreType.DMA,
        pltpu.SemaphoreType.DMA,
    ]

    def body(table, i_h, f_h, z128, out_p,
             ibufa, fbufa, ibufb, fbufb, dv1a, dv2a, dv1b, dv2b,
             t0, t1, t2, t3, acc, sem1, sem2, sems, semw1, semw2):
        tbufs = (t0, t1, t2, t3)

        c = lax.axis_index("c")
        s = lax.axis_index("s")
        wid = s * _NC + c

        pltpu.sync_copy(z128.at[pl.ds(s * chunk, chunk)],
                        acc.at[pl.ds(s * chunk, chunk)])
        if rem:
            @pl.when(s == _NS - 1)
            def _zero_rem():
                pltpu.sync_copy(z128.at[pl.ds(_NS * chunk, rem)],
                                acc.at[pl.ds(_NS * chunk, rem)])
        plsc.subcore_barrier()

        base_r = wid * nblk

        def compute_groups(fbuf, g_lo, g_hi):
            def g_body(g, carry2):
                bch = [fbuf[pl.ds(t * B + g * _L, _L)] for t in range(_S)]

                def l_body(l, carry3):
                    e = g * _L + l
                    lidx = jnp.zeros((_L,), jnp.int32) + l
                    bvs = [jnp.take(bch[t], lidx) for t in range(_S)]
                    for cch in range(_D // _L):
                        o = cch * _L
                        accv = None
                        for t in range(_S):
                            v = tbufs[t][e, pl.ds(o, _L)]
                            contrib = v * bvs[t]
                            accv = contrib if accv is None else accv + contrib
                        t0[e, pl.ds(o, _L)] = accv
                    return 0

                lax.fori_loop(0, _L, l_body, 0, unroll=4)
                return 0

            lax.fori_loop(g_lo, g_hi, g_body, 0)

        def extract_dst(ibuf, dv1, dv2):
            for i in range(G1):
                dv1[pl.ds(i * _L, _L)] = ibuf[pl.ds(_S * B + i * _L, _L)]
            for i in range(G - G1):
                dv2[pl.ds(i * _L, _L)] = ibuf[
                    pl.ds(_S * B + H1 + i * _L, _L)]

        def issue_half1(ibuf):
            return [pltpu.async_copy(table.at[ibuf.at[pl.ds(t * B, H1)]],
                                     tbufs[t].at[pl.ds(0, H1)], sem1)
                    for t in range(_S)]

        def issue_half2(ibuf):
            return [pltpu.async_copy(
                        table.at[ibuf.at[pl.ds(t * B + H1, B - H1)]],
                        tbufs[t].at[pl.ds(H1, B - H1)], sem2)
                    for t in range(_S)]

        def stage(j, ibuf, fbuf, inext, fnext, dv1, dv2, dv1n, dv2n,
                  pipelined):
            if pipelined:
                row2 = jnp.minimum(base_r + j + 1, e_total // B - 1)
                cpi = pltpu.async_copy(i_h.at[pl.ds(row2 * _IW, _IW)],
                                       inext, sems)
                cpf = pltpu.async_copy(f_h.at[pl.ds(row2 * _FW, _FW)],
                                       fnext, sems)
            for cp in _cps1[0]:
                cp.wait()
            compute_groups(fbuf, 0, G1)
            scw1 = pltpu.async_copy(t0.at[pl.ds(0, H1)], acc.at[dv1],
                                    add=True, sem=semw1)
            if pipelined:
                cpi.wait()
                cpf.wait()
            for cp in _cps2[0]:
                cp.wait()
            scw1.wait()
            if pipelined:
                _cps1[0] = issue_half1(inext)
            if _scw2[0] is not None:
                _scw2[0].wait()
                _scw2[0] = None
            compute_groups(fbuf, G1, G)
            scw2 = pltpu.async_copy(t0.at[pl.ds(H1, B - H1)], acc.at[dv2],
                                    add=True, sem=semw2)
            if pipelined:
                extract_dst(inext, dv1n, dv2n)
                _cps2[0] = issue_half2(inext)
                _scw2[0] = scw2
            else:
                scw2.wait()

        pltpu.sync_copy(i_h.at[pl.ds(base_r * _IW, _IW)], ibufa)
        pltpu.sync_copy(f_h.at[pl.ds(base_r * _FW, _FW)], fbufa)
        extract_dst(ibufa, dv1a, dv2a)
        _cps1 = [issue_half1(ibufa)]
        _cps2 = [issue_half2(ibufa)]
        _scw2 = [None]
        stage(0, ibufa, fbufa, ibufb, fbufb, dv1a, dv2a, dv1b, dv2b, True)

        def pair_body(j2, carry):
            j = j2 * 2 + 1
            stage(j, ibufb, fbufb, ibufa, fbufa,
                  dv1b, dv2b, dv1a, dv2a, True)
            stage(j + 1, ibufa, fbufa, ibufb, fbufb,
                  dv1a, dv2a, dv1b, dv2b, True)
            return 0

        lax.fori_loop(0, (nblk - 1) // 2, pair_body, 0)
        for cp in _cps1[0]:
            cp.wait()
        for cp in _cps2[0]:
            cp.wait()
        _scw2[0].wait()

        plsc.subcore_barrier()
        pltpu.sync_copy(acc.at[pl.ds(s * chunk, chunk)],
                        out_p.at[c, pl.ds(s * chunk, chunk)])
        if rem:
            @pl.when(s == _NS - 1)
            def _out_rem():
                pltpu.sync_copy(acc.at[pl.ds(_NS * chunk, rem)],
                                out_p.at[c, pl.ds(_NS * chunk, rem)])

    return pl.kernel(body, out_type=out_type, mesh=mesh,
                     scratch_types=scratch)


def _make_cnt(n, e_total):
    epw = e_total // _NW
    B = 80
    nblk = epw // B
    chunk = (n // _NS) & ~7
    rem = n - _NS * chunk

    mesh = plsc.VectorSubcoreMesh(core_axis_name="c", subcore_axis_name="s",
                                  num_cores=_NC, num_subcores=_NS)

    def body(dst_h, z16, out_c, dstv, ones, cacc, sem):
        c = lax.axis_index("c")
        s = lax.axis_index("s")
        wid = s * _NC + c

        pltpu.sync_copy(z16.at[pl.ds(s * chunk, chunk)],
                        cacc.at[pl.ds(s * chunk, chunk)])

        def ones_body(r, carry):
            for cch in range(_D // _L):
                ones[r, pl.ds(cch * _L, _L)] = jnp.zeros((_L,), jnp.float32) + 1.0
            return 0

        lax.fori_loop(0, B, ones_body, 0)
        if rem:
            @pl.when(s == _NS - 1)
            def _zero_rem():
                pltpu.sync_copy(z16.at[pl.ds(_NS * chunk, rem)],
                                cacc.at[pl.ds(_NS * chunk, rem)])
        plsc.subcore_barrier()

        base_e = wid * epw

        def block_body(j, carry):
            off = base_e + j * B
            pltpu.sync_copy(dst_h.at[pl.ds(off, B)], dstv)
            pltpu.sync_copy(ones, cacc.at[dstv], add=True)
            return 0

        lax.fori_loop(0, nblk, block_body, 0)

        plsc.subcore_barrier()
        pltpu.sync_copy(cacc.at[pl.ds(s * chunk, chunk)],
                        out_c.at[c, pl.ds(s * chunk, chunk)])
        if rem:
            @pl.when(s == _NS - 1)
            def _out_rem():
                pltpu.sync_copy(cacc.at[pl.ds(_NS * chunk, rem)],
                                out_c.at[c, pl.ds(_NS * chunk, rem)])

    return pl.kernel(
        body,
        out_type=[jax.ShapeDtypeStruct((_NC, n, _D), jnp.float32)],
        mesh=mesh,
        scratch_types=[
            pltpu.VMEM((B,), jnp.int32),
            pltpu.VMEM((B, _D), jnp.float32),
            pltpu.VMEM_SHARED((n, _D), jnp.float32),
            pltpu.SemaphoreType.DMA,
        ])


def _fin1_body(part_ref, cntp_ref, root_ref, b_ref, w_ref, o_ref):
    k = pl.program_id(1)
    aggsum = part_ref[0] + part_ref[1]
    cnt2 = cntp_ref[0] + cntp_ref[1]
    cnt = cnt2[:, 0:1]
    h = aggsum / jnp.maximum(cnt, 1.0) + root_ref[...] + b_ref[0]
    h = jnp.where(h > 0, h, jnp.exp(jnp.minimum(h, 0.0)) - 1.0)
    o_ref[...] = jnp.dot(h, w_ref[k], preferred_element_type=jnp.float32)


def _fin1proj2(part, cntp, root1t, b1, w2full, bn=400):
    n = root1t.shape[0]
    k1 = w2full.shape[0]
    nb = n // bn
    return pl.pallas_call(
        _fin1_body,
        grid=(nb, k1),
        in_specs=[
            pl.BlockSpec((_NC, bn, _D), lambda i, k: (0, i, 0)),
            pl.BlockSpec((_NC, bn, _D), lambda i, k: (0, i, 0)),
            pl.BlockSpec((bn, _D), lambda i, k: (i, 0)),
            pl.BlockSpec((1, _D), lambda i, k: (0, 0)),
            pl.BlockSpec((k1, _D, _D), lambda i, k: (0, 0, 0)),
        ],
        out_specs=pl.BlockSpec((bn, _D), lambda i, k: (k * nb + i, 0)),
        out_shape=jax.ShapeDtypeStruct((k1 * n, _D), jnp.float32),
    )(part, cntp, root1t, b1, w2full)


def _fin2_body(part_ref, cntp_ref, root_ref, b_ref, m1w_ref, m1b_ref,
               m2w_ref, m2b_ref, o_ref):
    aggsum = part_ref[0] + part_ref[1]
    cnt2 = cntp_ref[0] + cntp_ref[1]
    cnt = cnt2[:, 0:1]
    h = aggsum / jnp.maximum(cnt, 1.0) + root_ref[...] + b_ref[0]
    h = jnp.where(h > 0, h, jnp.exp(jnp.minimum(h, 0.0)) - 1.0)
    a = jnp.dot(h, m1w_ref[...], preferred_element_type=jnp.float32)
    a = jnp.maximum(a + m1b_ref[0], 0.0)
    o = jnp.dot(a, m2w_ref[...], preferred_element_type=jnp.float32)
    o_ref[...] = jnp.maximum(o + m2b_ref[0], 0.0)


def _fin2mlp(part, cntp, root2t, b2, m1w, m1b, m2w, m2b, bn=400):
    n = root2t.shape[0]
    co = m2w.shape[1]
    nb = n // bn
    return pl.pallas_call(
        _fin2_body,
        grid=(nb,),
        in_specs=[
            pl.BlockSpec((_NC, bn, _D), lambda i: (0, i, 0)),
            pl.BlockSpec((_NC, bn, _D), lambda i: (0, i, 0)),
            pl.BlockSpec((bn, _D), lambda i: (i, 0)),
            pl.BlockSpec((1, _D), lambda i: (0, 0)),
            pl.BlockSpec((_D, _D), lambda i: (0, 0)),
            pl.BlockSpec((1, _D), lambda i: (0, 0)),
            pl.BlockSpec((_D, co), lambda i: (0, 0)),
            pl.BlockSpec((1, co), lambda i: (0, 0)),
        ],
        out_specs=pl.BlockSpec((bn, co), lambda i: (i, 0)),
        out_shape=jax.ShapeDtypeStruct((n, co), jnp.float32),
    )(part, cntp, root2t, b2, m1w, m1b, m2w, m2b)


def kernel(x, edge_index, edge_attr, conv1_w, conv1_root, conv1_b,
           conv2_w, conv2_root, conv2_b, mlp1_w, mlp1_b, mlp2_w, mlp2_b):
    n = x.shape[0]
    e = edge_index.shape[1]
    k1 = conv1_w.shape[0]
    k2 = conv2_w.shape[0]

    src = edge_index[0]
    dst = edge_index[1]
    rows2d = e // _B
    a0 = edge_attr[:, 0].reshape(rows2d, _B)
    a1 = edge_attr[:, 1].reshape(rows2d, _B)
    src2 = src.reshape(rows2d, _B)
    dst2 = dst.reshape(rows2d, _B)

    i1s, f1s, i2s, f2s = _prep(n, e, a0, a1, src2, dst2)
    i1s = i1s.reshape(rows2d * _IW)
    f1s = f1s.reshape(rows2d * _FW)
    i2s = i2s.reshape(rows2d * _IW)
    f2s = f2s.reshape(rows2d * _FW)

    z128 = jnp.zeros((n, _D), jnp.float32)

    w1full = jnp.concatenate([conv1_w, conv1_root[None]], axis=0)
    y1 = _proj(x, w1full)
    root1t = lax.slice(y1, (k1 * n, 0), ((k1 + 1) * n, _D))

    (cntp,) = _make_cnt(n, e)(dst, z128)
    (part1,) = _make_agg(n, e)(y1, i1s, f1s, z128)

    w2full = jnp.concatenate([conv2_w, conv2_root[None]], axis=0)
    y2 = _fin1proj2(part1, cntp, root1t, conv1_b.reshape(1, _D), w2full)
    root2t = lax.slice(y2, (k2 * n, 0), ((k2 + 1) * n, _D))

    (part2,) = _make_agg(n, e)(y2, i2s, f2s, z128)

    out = _fin2mlp(part2, cntp, root2t, conv2_b.reshape(1, _D),
                   mlp1_w, mlp1_b.reshape(1, _D),
                   mlp2_w, mlp2_b.reshape(1, mlp2_b.shape[0]))
    return out


# speedup vs baseline: 18.3451x; 1.6154x over previous
import functools

import jax
import jax.numpy as jnp
from jax import lax
from jax.experimental import pallas as pl
from jax.experimental.pallas import tpu as pltpu
from jax.experimental.pallas import tpu_sc as plsc

_NC = 2
_NS = 16
_NW = _NC * _NS
_L = 16

_D = 128
_S = 4


_B = 80
_IW = 512
_FW = 384


def _prep_body(n, a0_ref, a1_ref, src_ref, dst_ref,
               i1_ref, f1_ref, i2_ref, f2_ref):
    a0 = a0_ref[...]
    a1 = a1_ref[...]
    srcv = src_ref[...]
    dstv = dst_ref[...]
    zi = jnp.zeros((a0.shape[0], _IW - 5 * _B), jnp.int32)
    zf = jnp.zeros((a0.shape[0], _FW - 4 * _B), jnp.float32)
    for ks, i_ref, f_ref in ((3, i1_ref, f1_ref), (5, i2_ref, f2_ref)):
        v0 = a0 * (ks - 1)
        bot0 = jnp.floor(v0)
        f0 = v0 - bot0
        i0 = bot0.astype(jnp.int32)
        v1 = a1 * (ks - 1)
        bot1 = jnp.floor(v1)
        f1 = v1 - bot1
        i1 = bot1.astype(jnp.int32)
        ws, rs = [], []
        for s in range(_S):
            bit0 = s & 1
            bit1 = (s >> 1) & 1
            w0 = f0 if bit0 else 1.0 - f0
            w1 = f1 if bit1 else 1.0 - f1
            idx0 = jnp.clip(i0 + bit0, 0, ks - 1)
            idx1 = jnp.clip(i1 + bit1, 0, ks - 1)
            wi = idx0 + ks * idx1
            ws.append(w0 * w1)
            rs.append(wi * n + srcv)
        i_ref[...] = jnp.concatenate(rs + [dstv, zi], axis=1)
        f_ref[...] = jnp.concatenate(ws + [zf], axis=1)


def _prep(n, e, a0, a1, src2, dst2):
    rows = a0.shape[0]
    grid = 5
    rb = rows // grid
    in_spec = pl.BlockSpec((rb, _B), lambda i: (i, 0))
    f32 = jnp.float32
    return pl.pallas_call(
        functools.partial(_prep_body, n),
        grid=(grid,),
        in_specs=[in_spec, in_spec, in_spec, in_spec],
        out_specs=[pl.BlockSpec((rb, _IW), lambda i: (i, 0)),
                   pl.BlockSpec((rb, _FW), lambda i: (i, 0)),
                   pl.BlockSpec((rb, _IW), lambda i: (i, 0)),
                   pl.BlockSpec((rb, _FW), lambda i: (i, 0))],
        out_shape=[
            jax.ShapeDtypeStruct((rows, _IW), jnp.int32),
            jax.ShapeDtypeStruct((rows, _FW), f32),
            jax.ShapeDtypeStruct((rows, _IW), jnp.int32),
            jax.ShapeDtypeStruct((rows, _FW), f32),
        ],
    )(a0, a1, src2, dst2)


def _proj_body(x_ref, w_ref, o_ref):
    k = pl.program_id(1)
    o_ref[...] = jnp.dot(x_ref[...], w_ref[k],
                         preferred_element_type=jnp.float32)


def _proj(xin, wfull, bn=2000):
    n = xin.shape[0]
    k1 = wfull.shape[0]
    nb = n // bn
    return pl.pallas_call(
        _proj_body,
        grid=(nb, k1),
        in_specs=[
            pl.BlockSpec((bn, _D), lambda i, k: (i, 0)),
            pl.BlockSpec((k1, _D, _D), lambda i, k: (0, 0, 0)),
        ],
        out_specs=pl.BlockSpec((bn, _D), lambda i, k: (k * nb + i, 0)),
        out_shape=jax.ShapeDtypeStruct((k1 * n, _D), jnp.float32),
    )(xin, wfull)


def _make_agg(n, e_total):
    epw = e_total // _NW
    B = 80
    nblk = epw // B
    G = B // _L
    chunk = (n // _NS) & ~7
    rem = n - _NS * chunk

    mesh = plsc.VectorSubcoreMesh(core_axis_name="c", subcore_axis_name="s",
                                  num_cores=_NC, num_subcores=_NS)

    out_type = [jax.ShapeDtypeStruct((_NC, n, _D), jnp.float32)]

    H1 = 48
    G1 = H1 // _L

    scratch = [
        pltpu.VMEM((_IW,), jnp.int32),
        pltpu.VMEM((_FW,), jnp.float32),
        pltpu.VMEM((_IW,), jnp.int32),
        pltpu.VMEM((_FW,), jnp.float32),
        pltpu.VMEM((H1,), jnp.int32),
        pltpu.VMEM((B - H1,), jnp.int32),
        pltpu.VMEM((H1,), jnp.int32),
        pltpu.VMEM((B - H1,), jnp.int32),
        pltpu.VMEM((B, _D), jnp.float32),
        pltpu.VMEM((B, _D), jnp.float32),
        pltpu.VMEM((B, _D), jnp.float32),
        pltpu.VMEM((B, _D), jnp.float32),
        pltpu.VMEM_SHARED((n, _D), jnp.float32),
        pltpu.SemaphoreType.DMA,
        pltpu.SemaphoreType.DMA,
        pltpu.SemaphoreType.DMA,
        pltpu.SemaphoreType.DMA,
        pltpu.SemaphoreType.DMA,
    ]

    def body(table, i_h, f_h, z128, out_p,
             ibufa, fbufa, ibufb, fbufb, dv1a, dv2a, dv1b, dv2b,
             t0, t1, t2, t3, acc, sem1, sem2, sems, semw1, semw2):
        tbufs = (t0, t1, t2, t3)

        c = lax.axis_index("c")
        s = lax.axis_index("s")
        wid = s * _NC + c

        pltpu.sync_copy(z128.at[pl.ds(s * chunk, chunk)],
                        acc.at[pl.ds(s * chunk, chunk)])
        if rem:
            @pl.when(s == _NS - 1)
            def _zero_rem():
                pltpu.sync_copy(z128.at[pl.ds(_NS * chunk, rem)],
                                acc.at[pl.ds(_NS * chunk, rem)])
        plsc.subcore_barrier()

        base_r = wid * nblk

        def compute_groups(fbuf, g_lo, g_hi):
            def g_body(g, carry2):
                bch = [fbuf[pl.ds(t * B + g * _L, _L)] for t in range(_S)]

                def l_body(l, carry3):
                    e = g * _L + l
                    lidx = jnp.zeros((_L,), jnp.int32) + l
                    bvs = [jnp.take(bch[t], lidx) for t in range(_S)]
                    for cch in range(_D // _L):
                        o = cch * _L
                        accv = None
                        for t in range(_S):
                            v = tbufs[t][e, pl.ds(o, _L)]
                            contrib = v * bvs[t]
                            accv = contrib if accv is None else accv + contrib
                        t0[e, pl.ds(o, _L)] = accv
                    return 0

                lax.fori_loop(0, _L, l_body, 0, unroll=8)
                return 0

            lax.fori_loop(g_lo, g_hi, g_body, 0)

        def extract_dst(ibuf, dv1, dv2):
            for i in range(G1):
                dv1[pl.ds(i * _L, _L)] = ibuf[pl.ds(_S * B + i * _L, _L)]
            for i in range(G - G1):
                dv2[pl.ds(i * _L, _L)] = ibuf[
                    pl.ds(_S * B + H1 + i * _L, _L)]

        def issue_half1(ibuf):
            return [pltpu.async_copy(table.at[ibuf.at[pl.ds(t * B, H1)]],
                                     tbufs[t].at[pl.ds(0, H1)], sem1)
                    for t in range(_S)]

        def issue_half2(ibuf):
            return [pltpu.async_copy(
                        table.at[ibuf.at[pl.ds(t * B + H1, B - H1)]],
                        tbufs[t].at[pl.ds(H1, B - H1)], sem2)
                    for t in range(_S)]

        def stage(j, ibuf, fbuf, inext, fnext, dv1, dv2, dv1n, dv2n,
                  pipelined):
            if pipelined:
                row2 = jnp.minimum(base_r + j + 1, e_total // B - 1)
                cpi = pltpu.async_copy(i_h.at[pl.ds(row2 * _IW, _IW)],
                                       inext, sems)
                cpf = pltpu.async_copy(f_h.at[pl.ds(row2 * _FW, _FW)],
                                       fnext, sems)
            for cp in _cps1[0]:
                cp.wait()
            compute_groups(fbuf, 0, G1)
            scw1 = pltpu.async_copy(t0.at[pl.ds(0, H1)], acc.at[dv1],
                                    add=True, sem=semw1)
            if pipelined:
                cpi.wait()
                cpf.wait()
            for cp in _cps2[0]:
                cp.wait()
            scw1.wait()
            if pipelined:
                _cps1[0] = issue_half1(inext)
            if _scw2[0] is not None:
                _scw2[0].wait()
                _scw2[0] = None
            compute_groups(fbuf, G1, G)
            scw2 = pltpu.async_copy(t0.at[pl.ds(H1, B - H1)], acc.at[dv2],
                                    add=True, sem=semw2)
            if pipelined:
                extract_dst(inext, dv1n, dv2n)
                _cps2[0] = issue_half2(inext)
                _scw2[0] = scw2
            else:
                scw2.wait()

        pltpu.sync_copy(i_h.at[pl.ds(base_r * _IW, _IW)], ibufa)
        pltpu.sync_copy(f_h.at[pl.ds(base_r * _FW, _FW)], fbufa)
        extract_dst(ibufa, dv1a, dv2a)
        _cps1 = [issue_half1(ibufa)]
        _cps2 = [issue_half2(ibufa)]
        _scw2 = [None]
        stage(0, ibufa, fbufa, ibufb, fbufb, dv1a, dv2a, dv1b, dv2b, True)

        def pair_body(j2, carry):
            j = j2 * 2 + 1
            stage(j, ibufb, fbufb, ibufa, fbufa,
                  dv1b, dv2b, dv1a, dv2a, True)
            stage(j + 1, ibufa, fbufa, ibufb, fbufb,
                  dv1a, dv2a, dv1b, dv2b, True)
            return 0

        lax.fori_loop(0, (nblk - 1) // 2, pair_body, 0)
        for cp in _cps1[0]:
            cp.wait()
        for cp in _cps2[0]:
            cp.wait()
        _scw2[0].wait()

        plsc.subcore_barrier()
        pltpu.sync_copy(acc.at[pl.ds(s * chunk, chunk)],
                        out_p.at[c, pl.ds(s * chunk, chunk)])
        if rem:
            @pl.when(s == _NS - 1)
            def _out_rem():
                pltpu.sync_copy(acc.at[pl.ds(_NS * chunk, rem)],
                                out_p.at[c, pl.ds(_NS * chunk, rem)])

    return pl.kernel(body, out_type=out_type, mesh=mesh,
                     scratch_types=scratch)


def _make_cnt(n, e_total):
    epw = e_total // _NW
    B = 80
    nblk = epw // B
    chunk = (n // _NS) & ~7
    rem = n - _NS * chunk

    mesh = plsc.VectorSubcoreMesh(core_axis_name="c", subcore_axis_name="s",
                                  num_cores=_NC, num_subcores=_NS)

    def body(dst_h, z16, out_c, dstv, ones, cacc, sem):
        c = lax.axis_index("c")
        s = lax.axis_index("s")
        wid = s * _NC + c

        pltpu.sync_copy(z16.at[pl.ds(s * chunk, chunk)],
                        cacc.at[pl.ds(s * chunk, chunk)])

        def ones_body(r, carry):
            for cch in range(_D // _L):
                ones[r, pl.ds(cch * _L, _L)] = jnp.zeros((_L,), jnp.float32) + 1.0
            return 0

        lax.fori_loop(0, B, ones_body, 0)
        if rem:
            @pl.when(s == _NS - 1)
            def _zero_rem():
                pltpu.sync_copy(z16.at[pl.ds(_NS * chunk, rem)],
                                cacc.at[pl.ds(_NS * chunk, rem)])
        plsc.subcore_barrier()

        base_e = wid * epw

        def block_body(j, carry):
            off = base_e + j * B
            pltpu.sync_copy(dst_h.at[pl.ds(off, B)], dstv)
            pltpu.sync_copy(ones, cacc.at[dstv], add=True)
            return 0

        lax.fori_loop(0, nblk, block_body, 0)

        plsc.subcore_barrier()
        pltpu.sync_copy(cacc.at[pl.ds(s * chunk, chunk)],
                        out_c.at[c, pl.ds(s * chunk, chunk)])
        if rem:
            @pl.when(s == _NS - 1)
            def _out_rem():
                pltpu.sync_copy(cacc.at[pl.ds(_NS * chunk, rem)],
                                out_c.at[c, pl.ds(_NS * chunk, rem)])

    return pl.kernel(
        body,
        out_type=[jax.ShapeDtypeStruct((_NC, n, _D), jnp.float32)],
        mesh=mesh,
        scratch_types=[
            pltpu.VMEM((B,), jnp.int32),
            pltpu.VMEM((B, _D), jnp.float32),
            pltpu.VMEM_SHARED((n, _D), jnp.float32),
            pltpu.SemaphoreType.DMA,
        ])


def _fin1_body(part_ref, cntp_ref, root_ref, b_ref, w_ref, o_ref):
    k = pl.program_id(1)
    aggsum = part_ref[0] + part_ref[1]
    cnt2 = cntp_ref[0] + cntp_ref[1]
    cnt = cnt2[:, 0:1]
    h = aggsum / jnp.maximum(cnt, 1.0) + root_ref[...] + b_ref[0]
    h = jnp.where(h > 0, h, jnp.exp(jnp.minimum(h, 0.0)) - 1.0)
    o_ref[...] = jnp.dot(h, w_ref[k], preferred_element_type=jnp.float32)


def _fin1proj2(part, cntp, root1t, b1, w2full, bn=2000):
    n = root1t.shape[0]
    k1 = w2full.shape[0]
    nb = n // bn
    return pl.pallas_call(
        _fin1_body,
        grid=(nb, k1),
        in_specs=[
            pl.BlockSpec((_NC, bn, _D), lambda i, k: (0, i, 0)),
            pl.BlockSpec((_NC, bn, _D), lambda i, k: (0, i, 0)),
            pl.BlockSpec((bn, _D), lambda i, k: (i, 0)),
            pl.BlockSpec((1, _D), lambda i, k: (0, 0)),
            pl.BlockSpec((k1, _D, _D), lambda i, k: (0, 0, 0)),
        ],
        out_specs=pl.BlockSpec((bn, _D), lambda i, k: (k * nb + i, 0)),
        out_shape=jax.ShapeDtypeStruct((k1 * n, _D), jnp.float32),
    )(part, cntp, root1t, b1, w2full)


def _fin2_body(part_ref, cntp_ref, root_ref, b_ref, m1w_ref, m1b_ref,
               m2w_ref, m2b_ref, o_ref):
    aggsum = part_ref[0] + part_ref[1]
    cnt2 = cntp_ref[0] + cntp_ref[1]
    cnt = cnt2[:, 0:1]
    h = aggsum / jnp.maximum(cnt, 1.0) + root_ref[...] + b_ref[0]
    h = jnp.where(h > 0, h, jnp.exp(jnp.minimum(h, 0.0)) - 1.0)
    a = jnp.dot(h, m1w_ref[...], preferred_element_type=jnp.float32)
    a = jnp.maximum(a + m1b_ref[0], 0.0)
    o = jnp.dot(a, m2w_ref[...], preferred_element_type=jnp.float32)
    o_ref[...] = jnp.maximum(o + m2b_ref[0], 0.0)


def _fin2mlp(part, cntp, root2t, b2, m1w, m1b, m2w, m2b, bn=2000):
    n = root2t.shape[0]
    co = m2w.shape[1]
    nb = n // bn
    return pl.pallas_call(
        _fin2_body,
        grid=(nb,),
        in_specs=[
            pl.BlockSpec((_NC, bn, _D), lambda i: (0, i, 0)),
            pl.BlockSpec((_NC, bn, _D), lambda i: (0, i, 0)),
            pl.BlockSpec((bn, _D), lambda i: (i, 0)),
            pl.BlockSpec((1, _D), lambda i: (0, 0)),
            pl.BlockSpec((_D, _D), lambda i: (0, 0)),
            pl.BlockSpec((1, _D), lambda i: (0, 0)),
            pl.BlockSpec((_D, co), lambda i: (0, 0)),
            pl.BlockSpec((1, co), lambda i: (0, 0)),
        ],
        out_specs=pl.BlockSpec((bn, co), lambda i: (i, 0)),
        out_shape=jax.ShapeDtypeStruct((n, co), jnp.float32),
    )(part, cntp, root2t, b2, m1w, m1b, m2w, m2b)


def kernel(x, edge_index, edge_attr, conv1_w, conv1_root, conv1_b,
           conv2_w, conv2_root, conv2_b, mlp1_w, mlp1_b, mlp2_w, mlp2_b):
    n = x.shape[0]
    e = edge_index.shape[1]
    k1 = conv1_w.shape[0]
    k2 = conv2_w.shape[0]

    src = edge_index[0]
    dst = edge_index[1]
    rows2d = e // _B
    a0 = edge_attr[:, 0].reshape(rows2d, _B)
    a1 = edge_attr[:, 1].reshape(rows2d, _B)
    src2 = src.reshape(rows2d, _B)
    dst2 = dst.reshape(rows2d, _B)

    i1s, f1s, i2s, f2s = _prep(n, e, a0, a1, src2, dst2)
    i1s = i1s.reshape(rows2d * _IW)
    f1s = f1s.reshape(rows2d * _FW)
    i2s = i2s.reshape(rows2d * _IW)
    f2s = f2s.reshape(rows2d * _FW)

    z128 = jnp.zeros((n, _D), jnp.float32)

    w1full = jnp.concatenate([conv1_w, conv1_root[None]], axis=0)
    y1 = _proj(x, w1full)
    root1t = lax.slice(y1, (k1 * n, 0), ((k1 + 1) * n, _D))

    (cntp,) = _make_cnt(n, e)(dst, z128)
    (part1,) = _make_agg(n, e)(y1, i1s, f1s, z128)

    w2full = jnp.concatenate([conv2_w, conv2_root[None]], axis=0)
    y2 = _fin1proj2(part1, cntp, root1t, conv1_b.reshape(1, _D), w2full)
    root2t = lax.slice(y2, (k2 * n, 0), ((k2 + 1) * n, _D))

    (part2,) = _make_agg(n, e)(y2, i2s, f2s, z128)

    out = _fin2mlp(part2, cntp, root2t, conv2_b.reshape(1, _D),
                   mlp1_w, mlp1_b.reshape(1, _D),
                   mlp2_w, mlp2_b.reshape(1, mlp2_b.shape[0]))
    return out


# speedup vs baseline: 18.9343x; 1.0321x over previous
import functools

import jax
import jax.numpy as jnp
from jax import lax
from jax.experimental import pallas as pl
from jax.experimental.pallas import tpu as pltpu
from jax.experimental.pallas import tpu_sc as plsc

_NC = 2
_NS = 16
_NW = _NC * _NS
_L = 16

_D = 128
_S = 4


_B = 80
_IW = 512
_FW = 384


def _prep_body(n, a0_ref, a1_ref, src_ref, dst_ref,
               i1_ref, f1_ref, i2_ref, f2_ref):
    a0 = a0_ref[...]
    a1 = a1_ref[...]
    srcv = src_ref[...]
    dstv = dst_ref[...]
    zi = jnp.zeros((a0.shape[0], _IW - 5 * _B), jnp.int32)
    zf = jnp.zeros((a0.shape[0], _FW - 4 * _B), jnp.float32)
    for ks, i_ref, f_ref in ((3, i1_ref, f1_ref), (5, i2_ref, f2_ref)):
        v0 = a0 * (ks - 1)
        bot0 = jnp.floor(v0)
        f0 = v0 - bot0
        i0 = bot0.astype(jnp.int32)
        v1 = a1 * (ks - 1)
        bot1 = jnp.floor(v1)
        f1 = v1 - bot1
        i1 = bot1.astype(jnp.int32)
        ws, rs = [], []
        for s in range(_S):
            bit0 = s & 1
            bit1 = (s >> 1) & 1
            w0 = f0 if bit0 else 1.0 - f0
            w1 = f1 if bit1 else 1.0 - f1
            idx0 = jnp.clip(i0 + bit0, 0, ks - 1)
            idx1 = jnp.clip(i1 + bit1, 0, ks - 1)
            wi = idx0 + ks * idx1
            ws.append(w0 * w1)
            rs.append(wi * n + srcv)
        i_ref[...] = jnp.concatenate(rs + [dstv, zi], axis=1)
        f_ref[...] = jnp.concatenate(ws + [zf], axis=1)


def _prep(n, e, a0, a1, src2, dst2):
    rows = a0.shape[0]
    grid = 5
    rb = rows // grid
    in_spec = pl.BlockSpec((rb, _B), lambda i: (i, 0))
    f32 = jnp.float32
    return pl.pallas_call(
        functools.partial(_prep_body, n),
        grid=(grid,),
        in_specs=[in_spec, in_spec, in_spec, in_spec],
        out_specs=[pl.BlockSpec((rb, _IW), lambda i: (i, 0)),
                   pl.BlockSpec((rb, _FW), lambda i: (i, 0)),
                   pl.BlockSpec((rb, _IW), lambda i: (i, 0)),
                   pl.BlockSpec((rb, _FW), lambda i: (i, 0))],
        out_shape=[
            jax.ShapeDtypeStruct((rows, _IW), jnp.int32),
            jax.ShapeDtypeStruct((rows, _FW), f32),
            jax.ShapeDtypeStruct((rows, _IW), jnp.int32),
            jax.ShapeDtypeStruct((rows, _FW), f32),
        ],
    )(a0, a1, src2, dst2)


def _proj_body(x_ref, w_ref, o_ref):
    k = pl.program_id(1)
    o_ref[...] = jnp.dot(x_ref[...], w_ref[k],
                         preferred_element_type=jnp.float32)


def _proj(xin, wfull, bn=10000):
    n = xin.shape[0]
    k1 = wfull.shape[0]
    nb = n // bn
    return pl.pallas_call(
        _proj_body,
        grid=(nb, k1),
        in_specs=[
            pl.BlockSpec((bn, _D), lambda i, k: (i, 0)),
            pl.BlockSpec((k1, _D, _D), lambda i, k: (0, 0, 0)),
        ],
        out_specs=pl.BlockSpec((bn, _D), lambda i, k: (k * nb + i, 0)),
        out_shape=jax.ShapeDtypeStruct((k1 * n, _D), jnp.float32),
    )(xin, wfull)


def _make_agg(n, e_total):
    epw = e_total // _NW
    B = 80
    nblk = epw // B
    G = B // _L
    chunk = (n // _NS) & ~7
    rem = n - _NS * chunk

    mesh = plsc.VectorSubcoreMesh(core_axis_name="c", subcore_axis_name="s",
                                  num_cores=_NC, num_subcores=_NS)

    out_type = [jax.ShapeDtypeStruct((_NC, n, _D), jnp.float32)]

    H1 = 48
    G1 = H1 // _L

    scratch = [
        pltpu.VMEM((_IW,), jnp.int32),
        pltpu.VMEM((_FW,), jnp.float32),
        pltpu.VMEM((_IW,), jnp.int32),
        pltpu.VMEM((_FW,), jnp.float32),
        pltpu.VMEM((H1,), jnp.int32),
        pltpu.VMEM((B - H1,), jnp.int32),
        pltpu.VMEM((H1,), jnp.int32),
        pltpu.VMEM((B - H1,), jnp.int32),
        pltpu.VMEM((B, _D), jnp.float32),
        pltpu.VMEM((B, _D), jnp.float32),
        pltpu.VMEM((B, _D), jnp.float32),
        pltpu.VMEM((B, _D), jnp.float32),
        pltpu.VMEM_SHARED((n, _D), jnp.float32),
        pltpu.SemaphoreType.DMA,
        pltpu.SemaphoreType.DMA,
        pltpu.SemaphoreType.DMA,
        pltpu.SemaphoreType.DMA,
        pltpu.SemaphoreType.DMA,
    ]

    def body(table, i_h, f_h, z128, out_p,
             ibufa, fbufa, ibufb, fbufb, dv1a, dv2a, dv1b, dv2b,
             t0, t1, t2, t3, acc, sem1, sem2, sems, semw1, semw2):
        tbufs = (t0, t1, t2, t3)

        c = lax.axis_index("c")
        s = lax.axis_index("s")
        wid = s * _NC + c

        pltpu.sync_copy(z128.at[pl.ds(s * chunk, chunk)],
                        acc.at[pl.ds(s * chunk, chunk)])
        if rem:
            @pl.when(s == _NS - 1)
            def _zero_rem():
                pltpu.sync_copy(z128.at[pl.ds(_NS * chunk, rem)],
                                acc.at[pl.ds(_NS * chunk, rem)])
        plsc.subcore_barrier()

        base_r = wid * nblk

        def compute_groups(fbuf, g_lo, g_hi):
            def g_body(g, carry2):
                bch = [fbuf[pl.ds(t * B + g * _L, _L)] for t in range(_S)]

                def l_body(l, carry3):
                    e = g * _L + l
                    lidx = jnp.zeros((_L,), jnp.int32) + l
                    bvs = [jnp.take(bch[t], lidx) for t in range(_S)]
                    for cch in range(_D // _L):
                        o = cch * _L
                        accv = None
                        for t in range(_S):
                            v = tbufs[t][e, pl.ds(o, _L)]
                            contrib = v * bvs[t]
                            accv = contrib if accv is None else accv + contrib
                        t0[e, pl.ds(o, _L)] = accv
                    return 0

                lax.fori_loop(0, _L, l_body, 0, unroll=8)
                return 0

            lax.fori_loop(g_lo, g_hi, g_body, 0)

        def extract_dst(ibuf, dv1, dv2):
            for i in range(G1):
                dv1[pl.ds(i * _L, _L)] = ibuf[pl.ds(_S * B + i * _L, _L)]
            for i in range(G - G1):
                dv2[pl.ds(i * _L, _L)] = ibuf[
                    pl.ds(_S * B + H1 + i * _L, _L)]

        def issue_half1(ibuf):
            return [pltpu.async_copy(table.at[ibuf.at[pl.ds(t * B, H1)]],
                                     tbufs[t].at[pl.ds(0, H1)], sem1)
                    for t in range(_S)]

        def issue_half2(ibuf):
            return [pltpu.async_copy(
                        table.at[ibuf.at[pl.ds(t * B + H1, B - H1)]],
                        tbufs[t].at[pl.ds(H1, B - H1)], sem2)
                    for t in range(_S)]

        def stage(j, ibuf, fbuf, inext, fnext, dv1, dv2, dv1n, dv2n,
                  pipelined):
            if pipelined:
                row2 = jnp.minimum(base_r + j + 1, e_total // B - 1)
                cpi = pltpu.async_copy(i_h.at[pl.ds(row2 * _IW, _IW)],
                                       inext, sems)
                cpf = pltpu.async_copy(f_h.at[pl.ds(row2 * _FW, _FW)],
                                       fnext, sems)
            for cp in _cps1[0]:
                cp.wait()
            compute_groups(fbuf, 0, G1)
            scw1 = pltpu.async_copy(t0.at[pl.ds(0, H1)], acc.at[dv1],
                                    add=True, sem=semw1)
            if pipelined:
                cpi.wait()
                cpf.wait()
            for cp in _cps2[0]:
                cp.wait()
            scw1.wait()
            if pipelined:
                _cps1[0] = issue_half1(inext)
            if _scw2[0] is not None:
                _scw2[0].wait()
                _scw2[0] = None
            compute_groups(fbuf, G1, G)
            scw2 = pltpu.async_copy(t0.at[pl.ds(H1, B - H1)], acc.at[dv2],
                                    add=True, sem=semw2)
            if pipelined:
                extract_dst(inext, dv1n, dv2n)
                _cps2[0] = issue_half2(inext)
                _scw2[0] = scw2
            else:
                scw2.wait()

        pltpu.sync_copy(i_h.at[pl.ds(base_r * _IW, _IW)], ibufa)
        pltpu.sync_copy(f_h.at[pl.ds(base_r * _FW, _FW)], fbufa)
        extract_dst(ibufa, dv1a, dv2a)
        _cps1 = [issue_half1(ibufa)]
        _cps2 = [issue_half2(ibufa)]
        _scw2 = [None]
        stage(0, ibufa, fbufa, ibufb, fbufb, dv1a, dv2a, dv1b, dv2b, True)

        def pair_body(j2, carry):
            j = j2 * 2 + 1
            stage(j, ibufb, fbufb, ibufa, fbufa,
                  dv1b, dv2b, dv1a, dv2a, True)
            stage(j + 1, ibufa, fbufa, ibufb, fbufb,
                  dv1a, dv2a, dv1b, dv2b, True)
            return 0

        lax.fori_loop(0, (nblk - 1) // 2, pair_body, 0)
        for cp in _cps1[0]:
            cp.wait()
        for cp in _cps2[0]:
            cp.wait()
        _scw2[0].wait()

        plsc.subcore_barrier()
        pltpu.sync_copy(acc.at[pl.ds(s * chunk, chunk)],
                        out_p.at[c, pl.ds(s * chunk, chunk)])
        if rem:
            @pl.when(s == _NS - 1)
            def _out_rem():
                pltpu.sync_copy(acc.at[pl.ds(_NS * chunk, rem)],
                                out_p.at[c, pl.ds(_NS * chunk, rem)])

    return pl.kernel(body, out_type=out_type, mesh=mesh,
                     scratch_types=scratch)


def _make_cnt(n, e_total):
    epw = e_total // _NW
    B = 80
    nblk = epw // B
    chunk = (n // _NS) & ~7
    rem = n - _NS * chunk

    mesh = plsc.VectorSubcoreMesh(core_axis_name="c", subcore_axis_name="s",
                                  num_cores=_NC, num_subcores=_NS)

    def body(dst_h, z16, out_c, dstv, ones, cacc, sem):
        c = lax.axis_index("c")
        s = lax.axis_index("s")
        wid = s * _NC + c

        pltpu.sync_copy(z16.at[pl.ds(s * chunk, chunk)],
                        cacc.at[pl.ds(s * chunk, chunk)])

        def ones_body(r, carry):
            for cch in range(_D // _L):
                ones[r, pl.ds(cch * _L, _L)] = jnp.zeros((_L,), jnp.float32) + 1.0
            return 0

        lax.fori_loop(0, B, ones_body, 0)
        if rem:
            @pl.when(s == _NS - 1)
            def _zero_rem():
                pltpu.sync_copy(z16.at[pl.ds(_NS * chunk, rem)],
                                cacc.at[pl.ds(_NS * chunk, rem)])
        plsc.subcore_barrier()

        base_e = wid * epw

        def block_body(j, carry):
            off = base_e + j * B
            pltpu.sync_copy(dst_h.at[pl.ds(off, B)], dstv)
            pltpu.sync_copy(ones, cacc.at[dstv], add=True)
            return 0

        lax.fori_loop(0, nblk, block_body, 0)

        plsc.subcore_barrier()
        pltpu.sync_copy(cacc.at[pl.ds(s * chunk, chunk)],
                        out_c.at[c, pl.ds(s * chunk, chunk)])
        if rem:
            @pl.when(s == _NS - 1)
            def _out_rem():
                pltpu.sync_copy(cacc.at[pl.ds(_NS * chunk, rem)],
                                out_c.at[c, pl.ds(_NS * chunk, rem)])

    return pl.kernel(
        body,
        out_type=[jax.ShapeDtypeStruct((_NC, n, _D), jnp.float32)],
        mesh=mesh,
        scratch_types=[
            pltpu.VMEM((B,), jnp.int32),
            pltpu.VMEM((B, _D), jnp.float32),
            pltpu.VMEM_SHARED((n, _D), jnp.float32),
            pltpu.SemaphoreType.DMA,
        ])


def _fin1_body(part_ref, cntp_ref, root_ref, b_ref, w_ref, o_ref):
    k = pl.program_id(1)
    aggsum = part_ref[0] + part_ref[1]
    cnt2 = cntp_ref[0] + cntp_ref[1]
    cnt = cnt2[:, 0:1]
    h = aggsum / jnp.maximum(cnt, 1.0) + root_ref[...] + b_ref[0]
    h = jnp.where(h > 0, h, jnp.exp(jnp.minimum(h, 0.0)) - 1.0)
    o_ref[...] = jnp.dot(h, w_ref[k], preferred_element_type=jnp.float32)


def _fin1proj2(part, cntp, root1t, b1, w2full, bn=10000):
    n = root1t.shape[0]
    k1 = w2full.shape[0]
    nb = n // bn
    return pl.pallas_call(
        _fin1_body,
        grid=(nb, k1),
        in_specs=[
            pl.BlockSpec((_NC, bn, _D), lambda i, k: (0, i, 0)),
            pl.BlockSpec((_NC, bn, _D), lambda i, k: (0, i, 0)),
            pl.BlockSpec((bn, _D), lambda i, k: (i, 0)),
            pl.BlockSpec((1, _D), lambda i, k: (0, 0)),
            pl.BlockSpec((k1, _D, _D), lambda i, k: (0, 0, 0)),
        ],
        out_specs=pl.BlockSpec((bn, _D), lambda i, k: (k * nb + i, 0)),
        out_shape=jax.ShapeDtypeStruct((k1 * n, _D), jnp.float32),
    )(part, cntp, root1t, b1, w2full)


def _fin2_body(part_ref, cntp_ref, root_ref, b_ref, m1w_ref, m1b_ref,
               m2w_ref, m2b_ref, o_ref):
    aggsum = part_ref[0] + part_ref[1]
    cnt2 = cntp_ref[0] + cntp_ref[1]
    cnt = cnt2[:, 0:1]
    h = aggsum / jnp.maximum(cnt, 1.0) + root_ref[...] + b_ref[0]
    h = jnp.where(h > 0, h, jnp.exp(jnp.minimum(h, 0.0)) - 1.0)
    a = jnp.dot(h, m1w_ref[...], preferred_element_type=jnp.float32)
    a = jnp.maximum(a + m1b_ref[0], 0.0)
    o = jnp.dot(a, m2w_ref[...], preferred_element_type=jnp.float32)
    o_ref[...] = jnp.maximum(o + m2b_ref[0], 0.0)


def _fin2mlp(part, cntp, root2t, b2, m1w, m1b, m2w, m2b, bn=10000):
    n = root2t.shape[0]
    co = m2w.shape[1]
    nb = n // bn
    return pl.pallas_call(
        _fin2_body,
        grid=(nb,),
        in_specs=[
            pl.BlockSpec((_NC, bn, _D), lambda i: (0, i, 0)),
            pl.BlockSpec((_NC, bn, _D), lambda i: (0, i, 0)),
            pl.BlockSpec((bn, _D), lambda i: (i, 0)),
            pl.BlockSpec((1, _D), lambda i: (0, 0)),
            pl.BlockSpec((_D, _D), lambda i: (0, 0)),
            pl.BlockSpec((1, _D), lambda i: (0, 0)),
            pl.BlockSpec((_D, co), lambda i: (0, 0)),
            pl.BlockSpec((1, co), lambda i: (0, 0)),
        ],
        out_specs=pl.BlockSpec((bn, co), lambda i: (i, 0)),
        out_shape=jax.ShapeDtypeStruct((n, co), jnp.float32),
    )(part, cntp, root2t, b2, m1w, m1b, m2w, m2b)


def kernel(x, edge_index, edge_attr, conv1_w, conv1_root, conv1_b,
           conv2_w, conv2_root, conv2_b, mlp1_w, mlp1_b, mlp2_w, mlp2_b):
    n = x.shape[0]
    e = edge_index.shape[1]
    k1 = conv1_w.shape[0]
    k2 = conv2_w.shape[0]

    src = edge_index[0]
    dst = edge_index[1]
    rows2d = e // _B
    a0 = edge_attr[:, 0].reshape(rows2d, _B)
    a1 = edge_attr[:, 1].reshape(rows2d, _B)
    src2 = src.reshape(rows2d, _B)
    dst2 = dst.reshape(rows2d, _B)

    i1s, f1s, i2s, f2s = _prep(n, e, a0, a1, src2, dst2)
    i1s = i1s.reshape(rows2d * _IW)
    f1s = f1s.reshape(rows2d * _FW)
    i2s = i2s.reshape(rows2d * _IW)
    f2s = f2s.reshape(rows2d * _FW)

    z128 = jnp.zeros((n, _D), jnp.float32)

    w1full = jnp.concatenate([conv1_w, conv1_root[None]], axis=0)
    y1 = _proj(x, w1full)
    root1t = lax.slice(y1, (k1 * n, 0), ((k1 + 1) * n, _D))

    (cntp,) = _make_cnt(n, e)(dst, z128)
    (part1,) = _make_agg(n, e)(y1, i1s, f1s, z128)

    w2full = jnp.concatenate([conv2_w, conv2_root[None]], axis=0)
    y2 = _fin1proj2(part1, cntp, root1t, conv1_b.reshape(1, _D), w2full)
    root2t = lax.slice(y2, (k2 * n, 0), ((k2 + 1) * n, _D))

    (part2,) = _make_agg(n, e)(y2, i2s, f2s, z128)

    out = _fin2mlp(part2, cntp, root2t, conv2_b.reshape(1, _D),
                   mlp1_w, mlp1_b.reshape(1, _D),
                   mlp2_w, mlp2_b.reshape(1, mlp2_b.shape[0]))
    return out
